# baseline (speedup 1.0000x reference)
import functools

import jax
import jax.numpy as jnp
from jax import lax
from jax.experimental import pallas as pl
from jax.experimental.pallas import tpu as pltpu

H = 128
K = 4


def _mm(a, b):
    return jnp.dot(a, b, preferred_element_type=jnp.float32)


def _node_enc_body(x_ref, w1_ref, b1_ref, w2_ref, b2_ref, o_ref):
    h1 = jnp.maximum(_mm(x_ref[...], w1_ref[...]) + b1_ref[...], 0.0)
    o_ref[...] = _mm(h1, w2_ref[...]) + b2_ref[...]


def _node_enc(x, p):
    n, f = x.shape
    r = 1000
    return pl.pallas_call(
        _node_enc_body,
        grid=(n // r,),
        in_specs=[
            pl.BlockSpec((r, f), lambda i: (i, 0)),
            pl.BlockSpec((f, H), lambda i: (0, 0)),
            pl.BlockSpec((1, H), lambda i: (0, 0)),
            pl.BlockSpec((H, H), lambda i: (0, 0)),
            pl.BlockSpec((1, H), lambda i: (0, 0)),
        ],
        out_specs=pl.BlockSpec((r, H), lambda i: (i, 0)),
        out_shape=jax.ShapeDtypeStruct((n, H), jnp.float32),
    )(x, p["l1"]["W"], p["l1"]["b"][None], p["l2"]["W"], p["l2"]["b"][None])


def _edge_enc_body(ea_ref, w1_ref, b1_ref, w2_ref, b2_ref, o_ref):
    h1 = jnp.maximum(_mm(ea_ref[0], w1_ref[0]) + b1_ref[0], 0.0)
    o_ref[0] = _mm(h1, w2_ref[0]) + b2_ref[0]


def _edge_enc(ea_all, w1, b1, w2, b2):
    k, e, f = ea_all.shape
    be = 4000
    return pl.pallas_call(
        _edge_enc_body,
        grid=(k, e // be),
        in_specs=[
            pl.BlockSpec((1, be, f), lambda k_, i: (k_, i, 0)),
            pl.BlockSpec((1, f, H), lambda k_, i: (k_, 0, 0)),
            pl.BlockSpec((1, 1, H), lambda k_, i: (k_, 0, 0)),
            pl.BlockSpec((1, H, H), lambda k_, i: (k_, 0, 0)),
            pl.BlockSpec((1, 1, H), lambda k_, i: (k_, 0, 0)),
        ],
        out_specs=pl.BlockSpec((1, be, H), lambda k_, i: (k_, i, 0)),
        out_shape=jax.ShapeDtypeStruct((k, e, H), jnp.float32),
    )(ea_all, w1, b1, w2, b2)


def _edge_mlp_body(hs_ref, hd_ref, e_ref, ws_ref, wd_ref, we_ref, b1_ref,
                   w2_ref, b2_ref, o_ref):
    pre = (_mm(hs_ref[0], ws_ref[0]) + _mm(hd_ref[0], wd_ref[0])
           + _mm(e_ref[0], we_ref[0]) + b1_ref[0])
    h1 = jnp.maximum(pre, 0.0)
    o_ref[0] = _mm(h1, w2_ref[0]) + b2_ref[0]


def _edge_mlp(hs_all, hd_all, e_all, ws, wd, we, b1, w2, b2):
    k, e, _ = e_all.shape
    be = 4000
    h2 = 2 * H
    return pl.pallas_call(
        _edge_mlp_body,
        grid=(k, e // be),
        in_specs=[
            pl.BlockSpec((1, be, H), lambda k_, i: (k_, i, 0)),
            pl.BlockSpec((1, be, H), lambda k_, i: (k_, i, 0)),
            pl.BlockSpec((1, be, H), lambda k_, i: (k_, i, 0)),
            pl.BlockSpec((1, H, h2), lambda k_, i: (k_, 0, 0)),
            pl.BlockSpec((1, H, h2), lambda k_, i: (k_, 0, 0)),
            pl.BlockSpec((1, H, h2), lambda k_, i: (k_, 0, 0)),
            pl.BlockSpec((1, 1, h2), lambda k_, i: (k_, 0, 0)),
            pl.BlockSpec((1, h2, H), lambda k_, i: (k_, 0, 0)),
            pl.BlockSpec((1, 1, H), lambda k_, i: (k_, 0, 0)),
        ],
        out_specs=pl.BlockSpec((1, be, H), lambda k_, i: (k_, i, 0)),
        out_shape=jax.ShapeDtypeStruct((k, e, H), jnp.float32),
    )(hs_all, hd_all, e_all, ws, wd, we, b1, w2, b2)


def _node_upd_body(h_ref, agg_ref, wh_ref, wa_ref, b1_ref, w2_ref, b2_ref,
                   g_ref, bn_ref, o_ref):
    h = h_ref[...]
    pre = _mm(h, wh_ref[...]) + _mm(agg_ref[...], wa_ref[...]) + b1_ref[...]
    h1 = jnp.maximum(pre, 0.0)
    y = h + _mm(h1, w2_ref[...]) + b2_ref[...]
    mu = jnp.mean(y, axis=-1, keepdims=True)
    var = jnp.mean((y - mu) ** 2, axis=-1, keepdims=True)
    o_ref[...] = (y - mu) * jax.lax.rsqrt(var + 1e-5) * g_ref[...] + bn_ref[...]


def _node_update(h, agg, wh, wa, b1, w2, b2, g, bn):
    n = h.shape[0]
    r = 1000
    h2 = 2 * H
    return pl.pallas_call(
        _node_upd_body,
        grid=(n // r,),
        in_specs=[
            pl.BlockSpec((r, H), lambda i: (i, 0)),
            pl.BlockSpec((r, H), lambda i: (i, 0)),
            pl.BlockSpec((H, h2), lambda i: (0, 0)),
            pl.BlockSpec((H, h2), lambda i: (0, 0)),
            pl.BlockSpec((1, h2), lambda i: (0, 0)),
            pl.BlockSpec((h2, H), lambda i: (0, 0)),
            pl.BlockSpec((1, H), lambda i: (0, 0)),
            pl.BlockSpec((1, H), lambda i: (0, 0)),
            pl.BlockSpec((1, H), lambda i: (0, 0)),
        ],
        out_specs=pl.BlockSpec((r, H), lambda i: (i, 0)),
        out_shape=jax.ShapeDtypeStruct((n, H), jnp.float32),
    )(h, agg, wh, wa, b1, w2, b2, g, bn)


def _heads_body(h_ref, w1_ref, b1_ref, w2r_ref, b2_ref, np_ref, gm_ref):
    i = pl.program_id(0)
    h = h_ref[...]
    h1 = jnp.maximum(_mm(h, w1_ref[...]) + b1_ref[...], 0.0)
    np_ref[...] = jnp.sum(h1 * w2r_ref[...], axis=-1, keepdims=True) + b2_ref[...]
    bmax = jnp.max(h, axis=0, keepdims=True)

    @pl.when(i == 0)
    def _():
        gm_ref[...] = bmax

    @pl.when(i > 0)
    def _():
        gm_ref[...] = jnp.maximum(gm_ref[...], bmax)


def _heads(h, p):
    n = h.shape[0]
    r = 1000
    hh = H // 2
    return pl.pallas_call(
        _heads_body,
        grid=(n // r,),
        in_specs=[
            pl.BlockSpec((r, H), lambda i: (i, 0)),
            pl.BlockSpec((H, hh), lambda i: (0, 0)),
            pl.BlockSpec((1, hh), lambda i: (0, 0)),
            pl.BlockSpec((1, hh), lambda i: (0, 0)),
            pl.BlockSpec((1, 1), lambda i: (0, 0)),
        ],
        out_specs=[
            pl.BlockSpec((r, 1), lambda i: (i, 0)),
            pl.BlockSpec((1, H), lambda i: (0, 0)),
        ],
        out_shape=[
            jax.ShapeDtypeStruct((n, 1), jnp.float32),
            jax.ShapeDtypeStruct((1, H), jnp.float32),
        ],
    )(h, p["l1"]["W"], p["l1"]["b"][None], p["l2"]["W"].T, p["l2"]["b"][None])


def _graph_head_body(g_ref, w1_ref, b1_ref, w2r_ref, b2_ref, o_ref):
    h1 = jnp.maximum(_mm(g_ref[...], w1_ref[...]) + b1_ref[...], 0.0)
    o_ref[...] = jnp.sum(h1 * w2r_ref[...], axis=-1, keepdims=True) + b2_ref[...]


def _graph_head(g, p):
    hh = H // 2
    return pl.pallas_call(
        _graph_head_body,
        grid=(1,),
        in_specs=[
            pl.BlockSpec((1, H), lambda i: (0, 0)),
            pl.BlockSpec((H, hh), lambda i: (0, 0)),
            pl.BlockSpec((1, hh), lambda i: (0, 0)),
            pl.BlockSpec((1, hh), lambda i: (0, 0)),
            pl.BlockSpec((1, 1), lambda i: (0, 0)),
        ],
        out_specs=pl.BlockSpec((1, 1), lambda i: (0, 0)),
        out_shape=jax.ShapeDtypeStruct((1, 1), jnp.float32),
    )(g, p["l1"]["W"], p["l1"]["b"][None], p["l2"]["W"].T, p["l2"]["b"][None])


def kernel(x, ei0, ei1, ei2, ei3, ea0, ea1, ea2, ea3, params):
    ei = [ei0, ei1, ei2, ei3]
    ea_all = jnp.stack([ea0, ea1, ea2, ea3])

    h = _node_enc(x, params["node_enc"])

    ew1 = jnp.stack([p["l1"]["W"] for p in params["edge_encs"]])
    eb1 = jnp.stack([p["l1"]["b"][None] for p in params["edge_encs"]])
    ew2 = jnp.stack([p["l2"]["W"] for p in params["edge_encs"]])
    eb2 = jnp.stack([p["l2"]["b"][None] for p in params["edge_encs"]])
    e_all = _edge_enc(ea_all, ew1, eb1, ew2, eb2)

    src_all = jnp.stack([e[0] for e in ei])
    dst_all = jnp.stack([e[1] for e in ei])

    n = x.shape[0]
    for lp in params["layers"]:
        ws = jnp.stack([p["l1"]["W"][:H] for p in lp["edge_mlps"]])
        wd = jnp.stack([p["l1"]["W"][H:2 * H] for p in lp["edge_mlps"]])
        we = jnp.stack([p["l1"]["W"][2 * H:] for p in lp["edge_mlps"]])
        b1 = jnp.stack([p["l1"]["b"][None] for p in lp["edge_mlps"]])
        w2 = jnp.stack([p["l2"]["W"] for p in lp["edge_mlps"]])
        b2 = jnp.stack([p["l2"]["b"][None] for p in lp["edge_mlps"]])

        hs_all = h[src_all]
        hd_all = h[dst_all]
        m_all = _edge_mlp(hs_all, hd_all, e_all, ws, wd, we, b1, w2, b2)

        agg = jnp.zeros((n, H), jnp.float32)
        for k in range(K):
            agg = agg + jax.ops.segment_sum(m_all[k], dst_all[k], num_segments=n)

        nm = lp["node_mlp"]
        h = _node_update(
            h, agg,
            nm["l1"]["W"][:H], nm["l1"]["W"][H:], nm["l1"]["b"][None],
            nm["l2"]["W"], nm["l2"]["b"][None],
            lp["norm"]["g"][None], lp["norm"]["b"][None])

    node_pred, gmax = _heads(h, params["reg_head"])
    graph_pred = _graph_head(gmax, params["graph_head"])
    return (jnp.reshape(node_pred, (-1,)), jnp.reshape(graph_pred, (1,)))


# speedup vs baseline: 1.3781x; 1.3781x over previous
import functools

import jax
import jax.numpy as jnp
from jax import lax
from jax.experimental import pallas as pl
from jax.experimental.pallas import tpu as pltpu
from jax.experimental.pallas import tpu_sc as plsc

H = 128
K = 4
NW = 32


def _sc_gather(table, idx_all):
    n_lists, e = idx_all.shape
    idx_flat = jnp.reshape(idx_all, (-1,))
    ch = 320
    nch = e // ch
    iters = (nch + NW - 1) // NW
    mesh = plsc.VectorSubcoreMesh(core_axis_name="c", subcore_axis_name="s")

    @functools.partial(
        pl.kernel,
        out_type=jax.ShapeDtypeStruct((n_lists, e, H), jnp.float32),
        mesh=mesh,
        scratch_types=[
            pltpu.VMEM((ch,), jnp.int32),
            pltpu.VMEM((ch, H), jnp.float32),
            pltpu.SemaphoreType.DMA,
        ],
    )
    def gather_k(h_hbm, idx_hbm, out_hbm, idx_v, rows_v, sem):
        wid = lax.axis_index("s") * 2 + lax.axis_index("c")
        for j in range(n_lists):
            def body(i, carry):
                c = wid + NW * i

                @pl.when(c < nch)
                def _():
                    start = c * ch
                    pltpu.sync_copy(idx_hbm.at[pl.ds(j * e + start, ch)], idx_v)
                    pltpu.async_copy(h_hbm.at[idx_v], rows_v, sem).wait()
                    pltpu.sync_copy(rows_v, out_hbm.at[j, pl.ds(start, ch)])

                return carry

            lax.fori_loop(0, iters, body, 0)

    return gather_k(table, idx_flat)


def _mm(a, b):
    return jnp.dot(a, b, preferred_element_type=jnp.float32)


def _node_enc_body(x_ref, w1_ref, b1_ref, w2_ref, b2_ref, o_ref):
    h1 = jnp.maximum(_mm(x_ref[...], w1_ref[...]) + b1_ref[...], 0.0)
    o_ref[...] = _mm(h1, w2_ref[...]) + b2_ref[...]


def _node_enc(x, p):
    n, f = x.shape
    r = 1000
    return pl.pallas_call(
        _node_enc_body,
        grid=(n // r,),
        in_specs=[
            pl.BlockSpec((r, f), lambda i: (i, 0)),
            pl.BlockSpec((f, H), lambda i: (0, 0)),
            pl.BlockSpec((1, H), lambda i: (0, 0)),
            pl.BlockSpec((H, H), lambda i: (0, 0)),
            pl.BlockSpec((1, H), lambda i: (0, 0)),
        ],
        out_specs=pl.BlockSpec((r, H), lambda i: (i, 0)),
        out_shape=jax.ShapeDtypeStruct((n, H), jnp.float32),
    )(x, p["l1"]["W"], p["l1"]["b"][None], p["l2"]["W"], p["l2"]["b"][None])


def _edge_enc_body(ea_ref, w1_ref, b1_ref, w2_ref, b2_ref, o_ref):
    h1 = jnp.maximum(_mm(ea_ref[0], w1_ref[0]) + b1_ref[0], 0.0)
    o_ref[0] = _mm(h1, w2_ref[0]) + b2_ref[0]


def _edge_enc(ea_all, w1, b1, w2, b2):
    k, e, f = ea_all.shape
    be = 4000
    return pl.pallas_call(
        _edge_enc_body,
        grid=(k, e // be),
        in_specs=[
            pl.BlockSpec((1, be, f), lambda k_, i: (k_, i, 0)),
            pl.BlockSpec((1, f, H), lambda k_, i: (k_, 0, 0)),
            pl.BlockSpec((1, 1, H), lambda k_, i: (k_, 0, 0)),
            pl.BlockSpec((1, H, H), lambda k_, i: (k_, 0, 0)),
            pl.BlockSpec((1, 1, H), lambda k_, i: (k_, 0, 0)),
        ],
        out_specs=pl.BlockSpec((1, be, H), lambda k_, i: (k_, i, 0)),
        out_shape=jax.ShapeDtypeStruct((k, e, H), jnp.float32),
    )(ea_all, w1, b1, w2, b2)


def _edge_mlp_body(hs_ref, hd_ref, e_ref, ws_ref, wd_ref, we_ref, b1_ref,
                   w2_ref, b2_ref, o_ref):
    pre = (_mm(hs_ref[0], ws_ref[0]) + _mm(hd_ref[0], wd_ref[0])
           + _mm(e_ref[0], we_ref[0]) + b1_ref[0])
    h1 = jnp.maximum(pre, 0.0)
    o_ref[0] = _mm(h1, w2_ref[0]) + b2_ref[0]


def _edge_mlp(hs_all, hd_all, e_all, ws, wd, we, b1, w2, b2):
    k, e, _ = e_all.shape
    be = 4000
    h2 = 2 * H
    return pl.pallas_call(
        _edge_mlp_body,
        grid=(k, e // be),
        in_specs=[
            pl.BlockSpec((1, be, H), lambda k_, i: (k_, i, 0)),
            pl.BlockSpec((1, be, H), lambda k_, i: (k_, i, 0)),
            pl.BlockSpec((1, be, H), lambda k_, i: (k_, i, 0)),
            pl.BlockSpec((1, H, h2), lambda k_, i: (k_, 0, 0)),
            pl.BlockSpec((1, H, h2), lambda k_, i: (k_, 0, 0)),
            pl.BlockSpec((1, H, h2), lambda k_, i: (k_, 0, 0)),
            pl.BlockSpec((1, 1, h2), lambda k_, i: (k_, 0, 0)),
            pl.BlockSpec((1, h2, H), lambda k_, i: (k_, 0, 0)),
            pl.BlockSpec((1, 1, H), lambda k_, i: (k_, 0, 0)),
        ],
        out_specs=pl.BlockSpec((1, be, H), lambda k_, i: (k_, i, 0)),
        out_shape=jax.ShapeDtypeStruct((k, e, H), jnp.float32),
    )(hs_all, hd_all, e_all, ws, wd, we, b1, w2, b2)


def _node_upd_body(h_ref, agg_ref, wh_ref, wa_ref, b1_ref, w2_ref, b2_ref,
                   g_ref, bn_ref, o_ref):
    h = h_ref[...]
    pre = _mm(h, wh_ref[...]) + _mm(agg_ref[...], wa_ref[...]) + b1_ref[...]
    h1 = jnp.maximum(pre, 0.0)
    y = h + _mm(h1, w2_ref[...]) + b2_ref[...]
    mu = jnp.mean(y, axis=-1, keepdims=True)
    var = jnp.mean((y - mu) ** 2, axis=-1, keepdims=True)
    o_ref[...] = (y - mu) * jax.lax.rsqrt(var + 1e-5) * g_ref[...] + bn_ref[...]


def _node_update(h, agg, wh, wa, b1, w2, b2, g, bn):
    n = h.shape[0]
    r = 1000
    h2 = 2 * H
    return pl.pallas_call(
        _node_upd_body,
        grid=(n // r,),
        in_specs=[
            pl.BlockSpec((r, H), lambda i: (i, 0)),
            pl.BlockSpec((r, H), lambda i: (i, 0)),
            pl.BlockSpec((H, h2), lambda i: (0, 0)),
            pl.BlockSpec((H, h2), lambda i: (0, 0)),
            pl.BlockSpec((1, h2), lambda i: (0, 0)),
            pl.BlockSpec((h2, H), lambda i: (0, 0)),
            pl.BlockSpec((1, H), lambda i: (0, 0)),
            pl.BlockSpec((1, H), lambda i: (0, 0)),
            pl.BlockSpec((1, H), lambda i: (0, 0)),
        ],
        out_specs=pl.BlockSpec((r, H), lambda i: (i, 0)),
        out_shape=jax.ShapeDtypeStruct((n, H), jnp.float32),
    )(h, agg, wh, wa, b1, w2, b2, g, bn)


def _heads_body(h_ref, w1_ref, b1_ref, w2r_ref, b2_ref, np_ref, gm_ref):
    i = pl.program_id(0)
    h = h_ref[...]
    h1 = jnp.maximum(_mm(h, w1_ref[...]) + b1_ref[...], 0.0)
    np_ref[...] = jnp.sum(h1 * w2r_ref[...], axis=-1, keepdims=True) + b2_ref[...]
    bmax = jnp.max(h, axis=0, keepdims=True)

    @pl.when(i == 0)
    def _():
        gm_ref[...] = bmax

    @pl.when(i > 0)
    def _():
        gm_ref[...] = jnp.maximum(gm_ref[...], bmax)


def _heads(h, p):
    n = h.shape[0]
    r = 1000
    hh = H // 2
    return pl.pallas_call(
        _heads_body,
        grid=(n // r,),
        in_specs=[
            pl.BlockSpec((r, H), lambda i: (i, 0)),
            pl.BlockSpec((H, hh), lambda i: (0, 0)),
            pl.BlockSpec((1, hh), lambda i: (0, 0)),
            pl.BlockSpec((1, hh), lambda i: (0, 0)),
            pl.BlockSpec((1, 1), lambda i: (0, 0)),
        ],
        out_specs=[
            pl.BlockSpec((r, 1), lambda i: (i, 0)),
            pl.BlockSpec((1, H), lambda i: (0, 0)),
        ],
        out_shape=[
            jax.ShapeDtypeStruct((n, 1), jnp.float32),
            jax.ShapeDtypeStruct((1, H), jnp.float32),
        ],
    )(h, p["l1"]["W"], p["l1"]["b"][None], p["l2"]["W"].T, p["l2"]["b"][None])


def _graph_head_body(g_ref, w1_ref, b1_ref, w2r_ref, b2_ref, o_ref):
    h1 = jnp.maximum(_mm(g_ref[...], w1_ref[...]) + b1_ref[...], 0.0)
    o_ref[...] = jnp.sum(h1 * w2r_ref[...], axis=-1, keepdims=True) + b2_ref[...]


def _graph_head(g, p):
    hh = H // 2
    return pl.pallas_call(
        _graph_head_body,
        grid=(1,),
        in_specs=[
            pl.BlockSpec((1, H), lambda i: (0, 0)),
            pl.BlockSpec((H, hh), lambda i: (0, 0)),
            pl.BlockSpec((1, hh), lambda i: (0, 0)),
            pl.BlockSpec((1, hh), lambda i: (0, 0)),
            pl.BlockSpec((1, 1), lambda i: (0, 0)),
        ],
        out_specs=pl.BlockSpec((1, 1), lambda i: (0, 0)),
        out_shape=jax.ShapeDtypeStruct((1, 1), jnp.float32),
    )(g, p["l1"]["W"], p["l1"]["b"][None], p["l2"]["W"].T, p["l2"]["b"][None])


def kernel(x, ei0, ei1, ei2, ei3, ea0, ea1, ea2, ea3, params):
    ei = [ei0, ei1, ei2, ei3]
    ea_all = jnp.stack([ea0, ea1, ea2, ea3])

    h = _node_enc(x, params["node_enc"])

    ew1 = jnp.stack([p["l1"]["W"] for p in params["edge_encs"]])
    eb1 = jnp.stack([p["l1"]["b"][None] for p in params["edge_encs"]])
    ew2 = jnp.stack([p["l2"]["W"] for p in params["edge_encs"]])
    eb2 = jnp.stack([p["l2"]["b"][None] for p in params["edge_encs"]])
    e_all = _edge_enc(ea_all, ew1, eb1, ew2, eb2)

    src_all = jnp.stack([e[0] for e in ei])
    dst_all = jnp.stack([e[1] for e in ei])

    n = x.shape[0]
    for lp in params["layers"]:
        ws = jnp.stack([p["l1"]["W"][:H] for p in lp["edge_mlps"]])
        wd = jnp.stack([p["l1"]["W"][H:2 * H] for p in lp["edge_mlps"]])
        we = jnp.stack([p["l1"]["W"][2 * H:] for p in lp["edge_mlps"]])
        b1 = jnp.stack([p["l1"]["b"][None] for p in lp["edge_mlps"]])
        w2 = jnp.stack([p["l2"]["W"] for p in lp["edge_mlps"]])
        b2 = jnp.stack([p["l2"]["b"][None] for p in lp["edge_mlps"]])

        g_all = _sc_gather(h, jnp.concatenate([src_all, dst_all]))
        m_all = _edge_mlp(g_all[:K], g_all[K:], e_all, ws, wd, we, b1, w2, b2)

        agg = jnp.zeros((n, H), jnp.float32)
        for k in range(K):
            agg = agg + jax.ops.segment_sum(m_all[k], dst_all[k], num_segments=n)

        nm = lp["node_mlp"]
        h = _node_update(
            h, agg,
            nm["l1"]["W"][:H], nm["l1"]["W"][H:], nm["l1"]["b"][None],
            nm["l2"]["W"], nm["l2"]["b"][None],
            lp["norm"]["g"][None], lp["norm"]["b"][None])

    node_pred, gmax = _heads(h, params["reg_head"])
    graph_pred = _graph_head(gmax, params["graph_head"])
    return (jnp.reshape(node_pred, (-1,)), jnp.reshape(graph_pred, (1,)))


# speedup vs baseline: 1.9357x; 1.4045x over previous
import functools

import jax
import jax.numpy as jnp
from jax import lax
from jax.experimental import pallas as pl
from jax.experimental.pallas import tpu as pltpu
from jax.experimental.pallas import tpu_sc as plsc

H = 128
K = 4
NW = 32


def _sc_gather(table, idx_flat, n_lists):
    e = idx_flat.shape[0] // n_lists
    ch = 320
    nch = e // ch
    iters = (nch + NW - 1) // NW
    mesh = plsc.VectorSubcoreMesh(core_axis_name="c", subcore_axis_name="s")

    @functools.partial(
        pl.kernel,
        out_type=jax.ShapeDtypeStruct((n_lists, e, H), jnp.float32),
        mesh=mesh,
        scratch_types=[
            pltpu.VMEM((ch,), jnp.int32),
            pltpu.VMEM((ch, H), jnp.float32),
            pltpu.SemaphoreType.DMA,
        ],
    )
    def gather_k(h_hbm, idx_hbm, out_hbm, idx_v, rows_v, sem):
        wid = lax.axis_index("s") * 2 + lax.axis_index("c")
        for j in range(n_lists):
            def body(i, carry):
                c = wid + NW * i

                @pl.when(c < nch)
                def _():
                    start = c * ch
                    pltpu.sync_copy(idx_hbm.at[pl.ds(j * e + start, ch)], idx_v)
                    pltpu.async_copy(h_hbm.at[idx_v], rows_v, sem).wait()
                    pltpu.sync_copy(rows_v, out_hbm.at[j, pl.ds(start, ch)])

                return carry

            lax.fori_loop(0, iters, body, 0)

    return gather_k(table, idx_flat)


NT = 320
SCH = 256


def _sc_scatter(m_all, ld_flat, bounds_flat, zrows):
    k4, e, _ = m_all.shape
    npad = NW * NT
    mesh = plsc.VectorSubcoreMesh(core_axis_name="c", subcore_axis_name="s")

    @functools.partial(
        pl.kernel,
        out_type=jax.ShapeDtypeStruct((npad * H,), jnp.float32),
        mesh=mesh,
        compiler_params=pltpu.CompilerParams(needs_layout_passes=False),
        scratch_types=[
            pltpu.VMEM(((NT + 1) * H,), jnp.float32),
            pltpu.VMEM((SCH, H), jnp.float32),
            pltpu.VMEM((SCH,), jnp.int32),
            pltpu.VMEM((48,), jnp.int32),
            pltpu.SemaphoreType.DMA,
        ],
    )
    def scatter_k(m_hbm, ld_hbm, b_hbm, z_hbm, out_hbm,
                  acc_v, mrow_v, ld_v, b_v, sem):
        wid = lax.axis_index("s") * 2 + lax.axis_index("c")
        pltpu.sync_copy(z_hbm, acc_v)
        lanes = lax.broadcasted_iota(jnp.int32, (16,), 0)
        cols = [lanes + 16 * j for j in range(H // 16)]

        def _bound(pos):
            acc = jnp.zeros((), jnp.int32)
            for rg in range(3):
                v = b_v[pl.ds(16 * rg, 16)]
                msk = (lanes + 16 * rg == pos).astype(jnp.int32)
                acc = acc + jnp.sum(v * msk)
            return acc

        for k in range(k4):
            pltpu.sync_copy(b_hbm.at[pl.ds(k * 48, 48)], b_v)
            s = _bound(wid)
            e_ = _bound(wid + 1)
            b0 = (s // 8) * 8
            nch = (e_ - b0 + SCH - 1) // SCH

            def chunk_body(i, carry, k=k, s=s, e_=e_, b0=b0):
                g0 = b0 + i * SCH
                g0c = jnp.minimum(g0, e - SCH)
                pltpu.sync_copy(m_hbm.at[k, pl.ds(g0c, SCH)], mrow_v)
                pltpu.sync_copy(ld_hbm.at[pl.ds(k * e + g0c, SCH)], ld_v)
                lo = jnp.maximum(g0, s)

                def grp_body(b, carry2):
                    lvec = ld_v[pl.ds(16 * b, 16)]
                    gbase = g0c + 16 * b
                    for rr in range(16):
                        valid = (gbase + rr >= lo) & (gbase + rr < e_)
                        dsc = lax.gather(
                            lvec, jnp.full((16, 1), rr, jnp.int32),
                            lax.GatherDimensionNumbers(
                                offset_dims=(), collapsed_slice_dims=(0,),
                                start_index_map=(0,)),
                            slice_sizes=(1,),
                            mode=lax.GatherScatterMode.PROMISE_IN_BOUNDS)
                        dvec = jnp.where(valid, dsc, NT) * H
                        for j in range(H // 16):
                            plsc.addupdate_scatter(
                                acc_v, [dvec + cols[j]],
                                mrow_v[16 * b + rr, pl.ds(16 * j, 16)])
                    return carry2

                lax.fori_loop(0, SCH // 16, grp_body, 0)
                return carry

            lax.fori_loop(0, nch, chunk_body, 0)

        pltpu.sync_copy(acc_v.at[pl.ds(0, NT * H)],
                        out_hbm.at[pl.ds(wid * NT * H, NT * H)])

    return jnp.reshape(scatter_k(m_all, ld_flat, bounds_flat, zrows),
                       (npad, H))


def _mm(a, b):
    return jnp.dot(a, b, preferred_element_type=jnp.float32)


def _node_enc_body(x_ref, w1_ref, b1_ref, w2_ref, b2_ref, o_ref):
    h1 = jnp.maximum(_mm(x_ref[...], w1_ref[...]) + b1_ref[...], 0.0)
    o_ref[...] = _mm(h1, w2_ref[...]) + b2_ref[...]


def _node_enc(x, p):
    n, f = x.shape
    r = 1024
    return pl.pallas_call(
        _node_enc_body,
        grid=(n // r,),
        in_specs=[
            pl.BlockSpec((r, f), lambda i: (i, 0)),
            pl.BlockSpec((f, H), lambda i: (0, 0)),
            pl.BlockSpec((1, H), lambda i: (0, 0)),
            pl.BlockSpec((H, H), lambda i: (0, 0)),
            pl.BlockSpec((1, H), lambda i: (0, 0)),
        ],
        out_specs=pl.BlockSpec((r, H), lambda i: (i, 0)),
        out_shape=jax.ShapeDtypeStruct((n, H), jnp.float32),
    )(x, p["l1"]["W"], p["l1"]["b"][None], p["l2"]["W"], p["l2"]["b"][None])


def _edge_enc_body(ea_ref, w1_ref, b1_ref, w2_ref, b2_ref, o_ref):
    h1 = jnp.maximum(_mm(ea_ref[0], w1_ref[0]) + b1_ref[0], 0.0)
    o_ref[0] = _mm(h1, w2_ref[0]) + b2_ref[0]


def _edge_enc(ea_all, w1, b1, w2, b2):
    k, e, f = ea_all.shape
    be = 4000
    return pl.pallas_call(
        _edge_enc_body,
        grid=(k, e // be),
        in_specs=[
            pl.BlockSpec((1, be, f), lambda k_, i: (k_, i, 0)),
            pl.BlockSpec((1, f, H), lambda k_, i: (k_, 0, 0)),
            pl.BlockSpec((1, 1, H), lambda k_, i: (k_, 0, 0)),
            pl.BlockSpec((1, H, H), lambda k_, i: (k_, 0, 0)),
            pl.BlockSpec((1, 1, H), lambda k_, i: (k_, 0, 0)),
        ],
        out_specs=pl.BlockSpec((1, be, H), lambda k_, i: (k_, i, 0)),
        out_shape=jax.ShapeDtypeStruct((k, e, H), jnp.float32),
    )(ea_all, w1, b1, w2, b2)


def _edge_mlp_body(hs_ref, hd_ref, e_ref, ws_ref, wd_ref, we_ref, b1_ref,
                   w2_ref, b2_ref, o_ref):
    pre = (_mm(hs_ref[0], ws_ref[0]) + _mm(hd_ref[0], wd_ref[0])
           + _mm(e_ref[0], we_ref[0]) + b1_ref[0])
    h1 = jnp.maximum(pre, 0.0)
    o_ref[0] = _mm(h1, w2_ref[0]) + b2_ref[0]


def _edge_mlp(g_all, e_all, ws, wd, we, b1, w2, b2):
    k, e, _ = e_all.shape
    be = 4000
    h2 = 2 * H
    return pl.pallas_call(
        _edge_mlp_body,
        grid=(k, e // be),
        in_specs=[
            pl.BlockSpec((1, be, H), lambda k_, i: (k_, i, 0)),
            pl.BlockSpec((1, be, H), lambda k_, i: (k_ + K, i, 0)),
            pl.BlockSpec((1, be, H), lambda k_, i: (k_, i, 0)),
            pl.BlockSpec((1, H, h2), lambda k_, i: (k_, 0, 0)),
            pl.BlockSpec((1, H, h2), lambda k_, i: (k_, 0, 0)),
            pl.BlockSpec((1, H, h2), lambda k_, i: (k_, 0, 0)),
            pl.BlockSpec((1, 1, h2), lambda k_, i: (k_, 0, 0)),
            pl.BlockSpec((1, h2, H), lambda k_, i: (k_, 0, 0)),
            pl.BlockSpec((1, 1, H), lambda k_, i: (k_, 0, 0)),
        ],
        out_specs=pl.BlockSpec((1, be, H), lambda k_, i: (k_, i, 0)),
        out_shape=jax.ShapeDtypeStruct((k, e, H), jnp.float32),
    )(g_all, g_all, e_all, ws, wd, we, b1, w2, b2)


def _node_upd_body(h_ref, agg_ref, wh_ref, wa_ref, b1_ref, w2_ref, b2_ref,
                   g_ref, bn_ref, o_ref):
    h = h_ref[...]
    pre = _mm(h, wh_ref[...]) + _mm(agg_ref[...], wa_ref[...]) + b1_ref[...]
    h1 = jnp.maximum(pre, 0.0)
    y = h + _mm(h1, w2_ref[...]) + b2_ref[...]
    mu = jnp.mean(y, axis=-1, keepdims=True)
    var = jnp.mean((y - mu) ** 2, axis=-1, keepdims=True)
    o_ref[...] = (y - mu) * jax.lax.rsqrt(var + 1e-5) * g_ref[...] + bn_ref[...]


def _node_update(h, agg, wh, wa, b1, w2, b2, g, bn):
    n = h.shape[0]
    r = 1024
    h2 = 2 * H
    return pl.pallas_call(
        _node_upd_body,
        grid=(n // r,),
        in_specs=[
            pl.BlockSpec((r, H), lambda i: (i, 0)),
            pl.BlockSpec((r, H), lambda i: (i, 0)),
            pl.BlockSpec((H, h2), lambda i: (0, 0)),
            pl.BlockSpec((H, h2), lambda i: (0, 0)),
            pl.BlockSpec((1, h2), lambda i: (0, 0)),
            pl.BlockSpec((h2, H), lambda i: (0, 0)),
            pl.BlockSpec((1, H), lambda i: (0, 0)),
            pl.BlockSpec((1, H), lambda i: (0, 0)),
            pl.BlockSpec((1, H), lambda i: (0, 0)),
        ],
        out_specs=pl.BlockSpec((r, H), lambda i: (i, 0)),
        out_shape=jax.ShapeDtypeStruct((n, H), jnp.float32),
    )(h, agg, wh, wa, b1, w2, b2, g, bn)


def _heads_body(n_real, h_ref, w1_ref, b1_ref, w2r_ref, b2_ref, np_ref, gm_ref):
    i = pl.program_id(0)
    h = h_ref[...]
    r = h.shape[0]
    h1 = jnp.maximum(_mm(h, w1_ref[...]) + b1_ref[...], 0.0)
    np_ref[...] = jnp.sum(h1 * w2r_ref[...], axis=-1, keepdims=True) + b2_ref[...]
    rid = lax.broadcasted_iota(jnp.int32, (r, 1), 0) + i * r
    bmax = jnp.max(jnp.where(rid < n_real, h, -jnp.inf), axis=0, keepdims=True)

    @pl.when(i == 0)
    def _():
        gm_ref[...] = bmax

    @pl.when(i > 0)
    def _():
        gm_ref[...] = jnp.maximum(gm_ref[...], bmax)


def _heads(h, p, n_real):
    n = h.shape[0]
    r = 1024
    hh = H // 2
    return pl.pallas_call(
        functools.partial(_heads_body, n_real),
        grid=(n // r,),
        in_specs=[
            pl.BlockSpec((r, H), lambda i: (i, 0)),
            pl.BlockSpec((H, hh), lambda i: (0, 0)),
            pl.BlockSpec((1, hh), lambda i: (0, 0)),
            pl.BlockSpec((1, hh), lambda i: (0, 0)),
            pl.BlockSpec((1, 1), lambda i: (0, 0)),
        ],
        out_specs=[
            pl.BlockSpec((r, 1), lambda i: (i, 0)),
            pl.BlockSpec((1, H), lambda i: (0, 0)),
        ],
        out_shape=[
            jax.ShapeDtypeStruct((n, 1), jnp.float32),
            jax.ShapeDtypeStruct((1, H), jnp.float32),
        ],
    )(h, p["l1"]["W"], p["l1"]["b"][None], p["l2"]["W"].T, p["l2"]["b"][None])


def _graph_head_body(g_ref, w1_ref, b1_ref, w2r_ref, b2_ref, o_ref):
    h1 = jnp.maximum(_mm(g_ref[...], w1_ref[...]) + b1_ref[...], 0.0)
    o_ref[...] = jnp.sum(h1 * w2r_ref[...], axis=-1, keepdims=True) + b2_ref[...]


def _graph_head(g, p):
    hh = H // 2
    return pl.pallas_call(
        _graph_head_body,
        grid=(1,),
        in_specs=[
            pl.BlockSpec((1, H), lambda i: (0, 0)),
            pl.BlockSpec((H, hh), lambda i: (0, 0)),
            pl.BlockSpec((1, hh), lambda i: (0, 0)),
            pl.BlockSpec((1, hh), lambda i: (0, 0)),
            pl.BlockSpec((1, 1), lambda i: (0, 0)),
        ],
        out_specs=pl.BlockSpec((1, 1), lambda i: (0, 0)),
        out_shape=jax.ShapeDtypeStruct((1, 1), jnp.float32),
    )(g, p["l1"]["W"], p["l1"]["b"][None], p["l2"]["W"].T, p["l2"]["b"][None])


def kernel(x, ei0, ei1, ei2, ei3, ea0, ea1, ea2, ea3, params):
    ei = [ei0, ei1, ei2, ei3]
    ea_all = jnp.stack([ea0, ea1, ea2, ea3])

    n = x.shape[0]
    npad = NW * NT
    x_pad = jnp.pad(x, ((0, npad - n), (0, 0)))
    h = _node_enc(x_pad, params["node_enc"])

    e = ei0.shape[1]
    src_all = jnp.stack([ee[0] for ee in ei])
    dst_all = jnp.stack([ee[1] for ee in ei])

    iota_e = jnp.arange(e, dtype=jnp.int32)
    skey = jnp.sort(dst_all * 131072 + iota_e[None], axis=-1)
    perm = skey & 131071
    dst_s = skey >> 17
    src_s = jnp.take_along_axis(src_all, perm, axis=1)
    ea_s = jnp.take_along_axis(
        jnp.reshape(ea_all, (K * e, -1)),
        jnp.reshape(perm + jnp.arange(K, dtype=jnp.int32)[:, None] * e,
                    (-1,))[:, None], axis=0)
    ea_s = jnp.reshape(ea_s, (K, e, -1))
    ld_flat = jnp.reshape(dst_s % NT, (-1,))
    targets = jnp.broadcast_to(jnp.arange(NW + 1, dtype=jnp.int32) * NT,
                               (K, NW + 1))
    bounds = jax.vmap(functools.partial(jnp.searchsorted, side="left"))(
        dst_s, targets).astype(jnp.int32)
    bounds_flat = jnp.reshape(
        jnp.pad(bounds, ((0, 0), (0, 48 - (NW + 1)))), (-1,))
    zrows = jnp.zeros(((NT + 1) * H,), jnp.float32)
    gidx_flat = jnp.reshape(jnp.concatenate([src_s, dst_s]), (-1,))

    ew1 = jnp.stack([p["l1"]["W"] for p in params["edge_encs"]])
    eb1 = jnp.stack([p["l1"]["b"][None] for p in params["edge_encs"]])
    ew2 = jnp.stack([p["l2"]["W"] for p in params["edge_encs"]])
    eb2 = jnp.stack([p["l2"]["b"][None] for p in params["edge_encs"]])
    e_all = _edge_enc(ea_s, ew1, eb1, ew2, eb2)

    for lp in params["layers"]:
        ws = jnp.stack([p["l1"]["W"][:H] for p in lp["edge_mlps"]])
        wd = jnp.stack([p["l1"]["W"][H:2 * H] for p in lp["edge_mlps"]])
        we = jnp.stack([p["l1"]["W"][2 * H:] for p in lp["edge_mlps"]])
        b1 = jnp.stack([p["l1"]["b"][None] for p in lp["edge_mlps"]])
        w2 = jnp.stack([p["l2"]["W"] for p in lp["edge_mlps"]])
        b2 = jnp.stack([p["l2"]["b"][None] for p in lp["edge_mlps"]])

        g_all = _sc_gather(h, gidx_flat, 2 * K)
        m_all = _edge_mlp(g_all, e_all, ws, wd, we, b1, w2, b2)

        agg_pad = _sc_scatter(m_all, ld_flat, bounds_flat, zrows)

        nm = lp["node_mlp"]
        h = _node_update(
            h, agg_pad,
            nm["l1"]["W"][:H], nm["l1"]["W"][H:], nm["l1"]["b"][None],
            nm["l2"]["W"], nm["l2"]["b"][None],
            lp["norm"]["g"][None], lp["norm"]["b"][None])

    node_pred, gmax = _heads(h, params["reg_head"], n)
    graph_pred = _graph_head(gmax, params["graph_head"])
    return (jnp.reshape(node_pred, (-1,))[:n], jnp.reshape(graph_pred, (1,)))


# speedup vs baseline: 2.0349x; 1.0513x over previous
import functools

import jax
import jax.numpy as jnp
from jax import lax
from jax.experimental import pallas as pl
from jax.experimental.pallas import tpu as pltpu
from jax.experimental.pallas import tpu_sc as plsc

H = 128
K = 4
NW = 32


HP = H // 2


def _sc_gather(table, idx_flat, n_lists):
    e = idx_flat.shape[0] // n_lists
    ch = 640
    nch = e // ch
    iters = (nch + NW - 1) // NW
    mesh = plsc.VectorSubcoreMesh(core_axis_name="c", subcore_axis_name="s")

    @functools.partial(
        pl.kernel,
        out_type=jax.ShapeDtypeStruct((n_lists, e, H), jnp.float32),
        mesh=mesh,
        scratch_types=[
            pltpu.VMEM((ch,), jnp.int32),
            pltpu.VMEM((ch, H), jnp.float32),
            pltpu.SemaphoreType.DMA,
        ],
    )
    def gather_k(h_hbm, idx_hbm, out_hbm, idx_v, rows_v, sem):
        wid = lax.axis_index("s") * 2 + lax.axis_index("c")
        for j in range(n_lists):
            def body(i, carry):
                c = wid + NW * i

                @pl.when(c < nch)
                def _():
                    start = c * ch
                    pltpu.sync_copy(idx_hbm.at[pl.ds(j * e + start, ch)], idx_v)
                    pltpu.async_copy(h_hbm.at[idx_v], rows_v, sem).wait()
                    pltpu.sync_copy(rows_v, out_hbm.at[j, pl.ds(start, ch)])

                return carry

            lax.fori_loop(0, iters, body, 0)

    return gather_k(table, idx_flat)


NT = 320
SCH = 256


def _sc_scatter(m_all, ld_flat, bounds_flat, zrows):
    k4, e, _ = m_all.shape
    npad = NW * NT
    mesh = plsc.VectorSubcoreMesh(core_axis_name="c", subcore_axis_name="s")

    @functools.partial(
        pl.kernel,
        out_type=jax.ShapeDtypeStruct((npad * H,), jnp.float32),
        mesh=mesh,
        compiler_params=pltpu.CompilerParams(needs_layout_passes=False),
        scratch_types=[
            pltpu.VMEM(((NT + 1) * H,), jnp.float32),
            pltpu.VMEM((SCH, H), jnp.float32),
            pltpu.VMEM((SCH,), jnp.int32),
            pltpu.VMEM((48,), jnp.int32),
            pltpu.SemaphoreType.DMA,
        ],
    )
    def scatter_k(m_hbm, ld_hbm, b_hbm, z_hbm, out_hbm,
                  acc_v, mrow_v, ld_v, b_v, sem):
        wid = lax.axis_index("s") * 2 + lax.axis_index("c")
        pltpu.sync_copy(z_hbm, acc_v)
        lanes = lax.broadcasted_iota(jnp.int32, (16,), 0)
        cols = [lanes + 16 * j for j in range(H // 16)]

        def _bound(pos):
            acc = jnp.zeros((), jnp.int32)
            for rg in range(3):
                v = b_v[pl.ds(16 * rg, 16)]
                msk = (lanes + 16 * rg == pos).astype(jnp.int32)
                acc = acc + jnp.sum(v * msk)
            return acc

        for k in range(k4):
            pltpu.sync_copy(b_hbm.at[pl.ds(k * 48, 48)], b_v)
            s = _bound(wid)
            e_ = _bound(wid + 1)
            b0 = (s // 8) * 8
            nch = (e_ - b0 + SCH - 1) // SCH

            def chunk_body(i, carry, k=k, s=s, e_=e_, b0=b0):
                g0 = b0 + i * SCH
                g0c = jnp.minimum(g0, e - SCH)
                pltpu.sync_copy(m_hbm.at[k, pl.ds(g0c, SCH)], mrow_v)
                pltpu.sync_copy(ld_hbm.at[pl.ds(k * e + g0c, SCH)], ld_v)
                lo = jnp.maximum(g0, s)

                def grp_body(b, carry2):
                    lvec = ld_v[pl.ds(16 * b, 16)]
                    gbase = g0c + 16 * b
                    for rr in range(16):
                        valid = (gbase + rr >= lo) & (gbase + rr < e_)
                        dsc = lax.gather(
                            lvec, jnp.full((16, 1), rr, jnp.int32),
                            lax.GatherDimensionNumbers(
                                offset_dims=(), collapsed_slice_dims=(0,),
                                start_index_map=(0,)),
                            slice_sizes=(1,),
                            mode=lax.GatherScatterMode.PROMISE_IN_BOUNDS)
                        dvec = jnp.where(valid, dsc, NT) * H
                        for j in range(H // 16):
                            plsc.addupdate_scatter(
                                acc_v, [dvec + cols[j]],
                                mrow_v[16 * b + rr, pl.ds(16 * j, 16)])
                    return carry2

                lax.fori_loop(0, SCH // 16, grp_body, 0)
                return carry

            lax.fori_loop(0, nch, chunk_body, 0)

        pltpu.sync_copy(acc_v.at[pl.ds(0, NT * H)],
                        out_hbm.at[pl.ds(wid * NT * H, NT * H)])

    return jnp.reshape(scatter_k(m_all, ld_flat, bounds_flat, zrows),
                       (npad, H))


def _mm(a, b):
    return jnp.dot(a, b, preferred_element_type=jnp.float32)


def _node_enc_body(x_ref, w1_ref, b1_ref, w2_ref, b2_ref, o_ref):
    h1 = jnp.maximum(_mm(x_ref[...], w1_ref[...]) + b1_ref[...], 0.0)
    o_ref[...] = _mm(h1, w2_ref[...]) + b2_ref[...]


def _node_enc(x, p):
    n, f = x.shape
    r = 1024
    return pl.pallas_call(
        _node_enc_body,
        grid=(n // r,),
        in_specs=[
            pl.BlockSpec((r, f), lambda i: (i, 0)),
            pl.BlockSpec((f, H), lambda i: (0, 0)),
            pl.BlockSpec((1, H), lambda i: (0, 0)),
            pl.BlockSpec((H, H), lambda i: (0, 0)),
            pl.BlockSpec((1, H), lambda i: (0, 0)),
        ],
        out_specs=pl.BlockSpec((r, H), lambda i: (i, 0)),
        out_shape=jax.ShapeDtypeStruct((n, H), jnp.float32),
    )(x, p["l1"]["W"], p["l1"]["b"][None], p["l2"]["W"], p["l2"]["b"][None])


def _edge_enc_body(ea_ref, w1_ref, b1_ref, w2_ref, b2_ref, o_ref):
    h1 = jnp.maximum(_mm(ea_ref[0], w1_ref[0]) + b1_ref[0], 0.0)
    o_ref[0] = (_mm(h1, w2_ref[0]) + b2_ref[0]).astype(jnp.bfloat16)


def _edge_enc(ea_all, w1, b1, w2, b2):
    k, e, f = ea_all.shape
    be = 4000
    return pl.pallas_call(
        _edge_enc_body,
        grid=(k, e // be),
        in_specs=[
            pl.BlockSpec((1, be, f), lambda k_, i: (k_, i, 0)),
            pl.BlockSpec((1, f, H), lambda k_, i: (k_, 0, 0)),
            pl.BlockSpec((1, 1, H), lambda k_, i: (k_, 0, 0)),
            pl.BlockSpec((1, H, H), lambda k_, i: (k_, 0, 0)),
            pl.BlockSpec((1, 1, H), lambda k_, i: (k_, 0, 0)),
        ],
        out_specs=pl.BlockSpec((1, be, H), lambda k_, i: (k_, i, 0)),
        out_shape=jax.ShapeDtypeStruct((k, e, H), jnp.bfloat16),
    )(ea_all, w1, b1, w2, b2)


def _edge_mlp_body(hs_ref, hd_ref, e_ref, ws_ref, wd_ref, we_ref, b1_ref,
                   w2_ref, b2_ref, o_ref):
    hs = hs_ref[0].astype(jnp.bfloat16)
    hd = hd_ref[0].astype(jnp.bfloat16)
    pre = (_mm(hs, ws_ref[0]) + _mm(hd, wd_ref[0])
           + _mm(e_ref[0], we_ref[0]) + b1_ref[0])
    h1 = jnp.maximum(pre, 0.0).astype(jnp.bfloat16)
    o_ref[0] = _mm(h1, w2_ref[0]) + b2_ref[0]


def _edge_mlp(g_all, e_all, ws, wd, we, b1, w2, b2):
    k, e, _ = e_all.shape
    be = 4000
    h2 = 2 * H
    return pl.pallas_call(
        _edge_mlp_body,
        grid=(k, e // be),
        in_specs=[
            pl.BlockSpec((1, be, H), lambda k_, i: (k_, i, 0)),
            pl.BlockSpec((1, be, H), lambda k_, i: (k_ + K, i, 0)),
            pl.BlockSpec((1, be, H), lambda k_, i: (k_, i, 0)),
            pl.BlockSpec((1, H, h2), lambda k_, i: (k_, 0, 0)),
            pl.BlockSpec((1, H, h2), lambda k_, i: (k_, 0, 0)),
            pl.BlockSpec((1, H, h2), lambda k_, i: (k_, 0, 0)),
            pl.BlockSpec((1, 1, h2), lambda k_, i: (k_, 0, 0)),
            pl.BlockSpec((1, h2, H), lambda k_, i: (k_, 0, 0)),
            pl.BlockSpec((1, 1, H), lambda k_, i: (k_, 0, 0)),
        ],
        out_specs=pl.BlockSpec((1, be, H), lambda k_, i: (k_, i, 0)),
        out_shape=jax.ShapeDtypeStruct((k, e, H), jnp.float32),
    )(g_all, g_all, e_all, ws, wd, we, b1, w2, b2)


def _node_upd_body(h_ref, agg_ref, wh_ref, wa_ref, b1_ref, w2_ref, b2_ref,
                   g_ref, bn_ref, o_ref):
    h = h_ref[...]
    pre = _mm(h, wh_ref[...]) + _mm(agg_ref[...], wa_ref[...]) + b1_ref[...]
    h1 = jnp.maximum(pre, 0.0)
    y = h + _mm(h1, w2_ref[...]) + b2_ref[...]
    mu = jnp.mean(y, axis=-1, keepdims=True)
    var = jnp.mean((y - mu) ** 2, axis=-1, keepdims=True)
    o_ref[...] = (y - mu) * jax.lax.rsqrt(var + 1e-5) * g_ref[...] + bn_ref[...]


def _node_update(h, agg, wh, wa, b1, w2, b2, g, bn):
    n = h.shape[0]
    r = 1024
    h2 = 2 * H
    return pl.pallas_call(
        _node_upd_body,
        grid=(n // r,),
        in_specs=[
            pl.BlockSpec((r, H), lambda i: (i, 0)),
            pl.BlockSpec((r, H), lambda i: (i, 0)),
            pl.BlockSpec((H, h2), lambda i: (0, 0)),
            pl.BlockSpec((H, h2), lambda i: (0, 0)),
            pl.BlockSpec((1, h2), lambda i: (0, 0)),
            pl.BlockSpec((h2, H), lambda i: (0, 0)),
            pl.BlockSpec((1, H), lambda i: (0, 0)),
            pl.BlockSpec((1, H), lambda i: (0, 0)),
            pl.BlockSpec((1, H), lambda i: (0, 0)),
        ],
        out_specs=pl.BlockSpec((r, H), lambda i: (i, 0)),
        out_shape=jax.ShapeDtypeStruct((n, H), jnp.float32),
    )(h, agg, wh, wa, b1, w2, b2, g, bn)


def _heads_body(n_real, h_ref, w1_ref, b1_ref, w2r_ref, b2_ref, np_ref, gm_ref):
    i = pl.program_id(0)
    h = h_ref[...]
    r = h.shape[0]
    h1 = jnp.maximum(_mm(h, w1_ref[...]) + b1_ref[...], 0.0)
    np_ref[...] = jnp.sum(h1 * w2r_ref[...], axis=-1, keepdims=True) + b2_ref[...]
    rid = lax.broadcasted_iota(jnp.int32, (r, 1), 0) + i * r
    bmax = jnp.max(jnp.where(rid < n_real, h, -jnp.inf), axis=0, keepdims=True)

    @pl.when(i == 0)
    def _():
        gm_ref[...] = bmax

    @pl.when(i > 0)
    def _():
        gm_ref[...] = jnp.maximum(gm_ref[...], bmax)


def _heads(h, p, n_real):
    n = h.shape[0]
    r = 1024
    hh = H // 2
    return pl.pallas_call(
        functools.partial(_heads_body, n_real),
        grid=(n // r,),
        in_specs=[
            pl.BlockSpec((r, H), lambda i: (i, 0)),
            pl.BlockSpec((H, hh), lambda i: (0, 0)),
            pl.BlockSpec((1, hh), lambda i: (0, 0)),
            pl.BlockSpec((1, hh), lambda i: (0, 0)),
            pl.BlockSpec((1, 1), lambda i: (0, 0)),
        ],
        out_specs=[
            pl.BlockSpec((r, 1), lambda i: (i, 0)),
            pl.BlockSpec((1, H), lambda i: (0, 0)),
        ],
        out_shape=[
            jax.ShapeDtypeStruct((n, 1), jnp.float32),
            jax.ShapeDtypeStruct((1, H), jnp.float32),
        ],
    )(h, p["l1"]["W"], p["l1"]["b"][None], p["l2"]["W"].T, p["l2"]["b"][None])


def _graph_head_body(g_ref, w1_ref, b1_ref, w2r_ref, b2_ref, o_ref):
    h1 = jnp.maximum(_mm(g_ref[...], w1_ref[...]) + b1_ref[...], 0.0)
    o_ref[...] = jnp.sum(h1 * w2r_ref[...], axis=-1, keepdims=True) + b2_ref[...]


def _graph_head(g, p):
    hh = H // 2
    return pl.pallas_call(
        _graph_head_body,
        grid=(1,),
        in_specs=[
            pl.BlockSpec((1, H), lambda i: (0, 0)),
            pl.BlockSpec((H, hh), lambda i: (0, 0)),
            pl.BlockSpec((1, hh), lambda i: (0, 0)),
            pl.BlockSpec((1, hh), lambda i: (0, 0)),
            pl.BlockSpec((1, 1), lambda i: (0, 0)),
        ],
        out_specs=pl.BlockSpec((1, 1), lambda i: (0, 0)),
        out_shape=jax.ShapeDtypeStruct((1, 1), jnp.float32),
    )(g, p["l1"]["W"], p["l1"]["b"][None], p["l2"]["W"].T, p["l2"]["b"][None])


def kernel(x, ei0, ei1, ei2, ei3, ea0, ea1, ea2, ea3, params):
    ei = [ei0, ei1, ei2, ei3]
    ea_all = jnp.stack([ea0, ea1, ea2, ea3])

    n = x.shape[0]
    npad = NW * NT
    x_pad = jnp.pad(x, ((0, npad - n), (0, 0)))
    h = _node_enc(x_pad, params["node_enc"])

    e = ei0.shape[1]
    src_all = jnp.stack([ee[0] for ee in ei])
    dst_all = jnp.stack([ee[1] for ee in ei])

    iota_e = jnp.arange(e, dtype=jnp.int32)
    skey = jnp.sort(dst_all * 131072 + iota_e[None], axis=-1)
    perm = skey & 131071
    dst_s = skey >> 17
    src_s = jnp.take_along_axis(src_all, perm, axis=1)
    ea_s = jnp.take_along_axis(
        jnp.reshape(ea_all, (K * e, -1)),
        jnp.reshape(perm + jnp.arange(K, dtype=jnp.int32)[:, None] * e,
                    (-1,))[:, None], axis=0)
    ea_s = jnp.reshape(ea_s, (K, e, -1))
    ld_flat = jnp.reshape(dst_s % NT, (-1,))
    targets = jnp.broadcast_to(jnp.arange(NW + 1, dtype=jnp.int32) * NT,
                               (K, NW + 1))
    bounds = jax.vmap(functools.partial(jnp.searchsorted, side="left"))(
        dst_s, targets).astype(jnp.int32)
    bounds_flat = jnp.reshape(
        jnp.pad(bounds, ((0, 0), (0, 48 - (NW + 1)))), (-1,))
    zrows = jnp.zeros(((NT + 1) * H,), jnp.float32)
    gidx_flat = jnp.reshape(jnp.concatenate([src_s, dst_s]), (-1,))

    ew1 = jnp.stack([p["l1"]["W"] for p in params["edge_encs"]])
    eb1 = jnp.stack([p["l1"]["b"][None] for p in params["edge_encs"]])
    ew2 = jnp.stack([p["l2"]["W"] for p in params["edge_encs"]])
    eb2 = jnp.stack([p["l2"]["b"][None] for p in params["edge_encs"]])
    e_all = _edge_enc(ea_s, ew1, eb1, ew2, eb2)

    for lp in params["layers"]:
        ws = jnp.stack([p["l1"]["W"][:H]
                        for p in lp["edge_mlps"]]).astype(jnp.bfloat16)
        wd = jnp.stack([p["l1"]["W"][H:2 * H]
                        for p in lp["edge_mlps"]]).astype(jnp.bfloat16)
        we = jnp.stack([p["l1"]["W"][2 * H:]
                        for p in lp["edge_mlps"]]).astype(jnp.bfloat16)
        b1 = jnp.stack([p["l1"]["b"][None] for p in lp["edge_mlps"]])
        w2 = jnp.stack([p["l2"]["W"]
                        for p in lp["edge_mlps"]]).astype(jnp.bfloat16)
        b2 = jnp.stack([p["l2"]["b"][None] for p in lp["edge_mlps"]])

        g_all = _sc_gather(h, gidx_flat, 2 * K)
        m_all = _edge_mlp(g_all, e_all, ws, wd, we, b1, w2, b2)

        agg_pad = _sc_scatter(m_all, ld_flat, bounds_flat, zrows)

        nm = lp["node_mlp"]
        h = _node_update(
            h, agg_pad,
            nm["l1"]["W"][:H], nm["l1"]["W"][H:], nm["l1"]["b"][None],
            nm["l2"]["W"], nm["l2"]["b"][None],
            lp["norm"]["g"][None], lp["norm"]["b"][None])

    node_pred, gmax = _heads(h, params["reg_head"], n)
    graph_pred = _graph_head(gmax, params["graph_head"])
    return (jnp.reshape(node_pred, (-1,))[:n], jnp.reshape(graph_pred, (1,)))


# speedup vs baseline: 2.2634x; 1.1123x over previous
import functools

import jax
import jax.numpy as jnp
from jax import lax
from jax.experimental import pallas as pl
from jax.experimental.pallas import tpu as pltpu
from jax.experimental.pallas import tpu_sc as plsc

H = 128
K = 4
NW = 32


HP = H // 2


def _sc_gather(table, idx_flat, n_lists):
    e = idx_flat.shape[0] // n_lists
    ch = 640
    nch = e // ch
    iters = (nch + NW - 1) // NW
    mesh = plsc.VectorSubcoreMesh(core_axis_name="c", subcore_axis_name="s")

    @functools.partial(
        pl.kernel,
        out_type=jax.ShapeDtypeStruct((n_lists, e, H), jnp.float32),
        mesh=mesh,
        scratch_types=[
            pltpu.VMEM((ch,), jnp.int32),
            pltpu.VMEM((ch, H), jnp.float32),
            pltpu.SemaphoreType.DMA,
        ],
    )
    def gather_k(h_hbm, idx_hbm, out_hbm, idx_v, rows_v, sem):
        wid = lax.axis_index("s") * 2 + lax.axis_index("c")
        for j in range(n_lists):
            def body(i, carry):
                c = wid + NW * i

                @pl.when(c < nch)
                def _():
                    start = c * ch
                    pltpu.sync_copy(idx_hbm.at[pl.ds(j * e + start, ch)], idx_v)
                    pltpu.async_copy(h_hbm.at[idx_v], rows_v, sem).wait()
                    pltpu.sync_copy(rows_v, out_hbm.at[j, pl.ds(start, ch)])

                return carry

            lax.fori_loop(0, iters, body, 0)

    return gather_k(table, idx_flat)


NT = 320
SCH = 256


def _sc_scatter(m_all, ld_flat, bounds_flat, init_flat, ztr):
    k4, e, _ = m_all.shape
    npad = NW * NT
    mesh = plsc.VectorSubcoreMesh(core_axis_name="c", subcore_axis_name="s")

    @functools.partial(
        pl.kernel,
        out_type=jax.ShapeDtypeStruct((npad * H,), jnp.float32),
        mesh=mesh,
        compiler_params=pltpu.CompilerParams(needs_layout_passes=False),
        scratch_types=[
            pltpu.VMEM(((NT + 1) * H,), jnp.float32),
            pltpu.VMEM((SCH, H), jnp.float32),
            pltpu.VMEM((SCH,), jnp.int32),
            pltpu.VMEM((48,), jnp.int32),
            pltpu.SemaphoreType.DMA,
        ],
    )
    def scatter_k(m_hbm, ld_hbm, b_hbm, z_hbm, ztr_hbm, out_hbm,
                  acc_v, mrow_v, ld_v, b_v, sem):
        wid = lax.axis_index("s") * 2 + lax.axis_index("c")
        pltpu.sync_copy(z_hbm.at[pl.ds(wid * NT * H, NT * H)],
                        acc_v.at[pl.ds(0, NT * H)])
        pltpu.sync_copy(ztr_hbm, acc_v.at[pl.ds(NT * H, H)])
        lanes = lax.broadcasted_iota(jnp.int32, (16,), 0)
        cols = [lanes + 16 * j for j in range(H // 16)]

        def _bound(pos):
            acc = jnp.zeros((), jnp.int32)
            for rg in range(3):
                v = b_v[pl.ds(16 * rg, 16)]
                msk = (lanes + 16 * rg == pos).astype(jnp.int32)
                acc = acc + jnp.sum(v * msk)
            return acc

        for k in range(k4):
            pltpu.sync_copy(b_hbm.at[pl.ds(k * 48, 48)], b_v)
            s = _bound(wid)
            e_ = _bound(wid + 1)
            b0 = (s // 8) * 8
            nch = (e_ - b0 + SCH - 1) // SCH

            def chunk_body(i, carry, k=k, s=s, e_=e_, b0=b0):
                g0 = b0 + i * SCH
                g0c = jnp.minimum(g0, e - SCH)
                pltpu.sync_copy(m_hbm.at[k, pl.ds(g0c, SCH)], mrow_v)
                pltpu.sync_copy(ld_hbm.at[pl.ds(k * e + g0c, SCH)], ld_v)
                lo = jnp.maximum(g0, s)

                def grp_body(b, carry2):
                    lvec = ld_v[pl.ds(16 * b, 16)]
                    gbase = g0c + 16 * b
                    for rr in range(16):
                        valid = (gbase + rr >= lo) & (gbase + rr < e_)
                        dsc = lax.gather(
                            lvec, jnp.full((16, 1), rr, jnp.int32),
                            lax.GatherDimensionNumbers(
                                offset_dims=(), collapsed_slice_dims=(0,),
                                start_index_map=(0,)),
                            slice_sizes=(1,),
                            mode=lax.GatherScatterMode.PROMISE_IN_BOUNDS)
                        dvec = jnp.where(valid, dsc, NT) * H
                        for j in range(H // 16):
                            plsc.addupdate_scatter(
                                acc_v, [dvec + cols[j]],
                                mrow_v[16 * b + rr, pl.ds(16 * j, 16)])
                    return carry2

                lax.fori_loop(0, SCH // 16, grp_body, 0)
                return carry

            lax.fori_loop(0, nch, chunk_body, 0)

        pltpu.sync_copy(acc_v.at[pl.ds(0, NT * H)],
                        out_hbm.at[pl.ds(wid * NT * H, NT * H)])

    return scatter_k(m_all, ld_flat, bounds_flat, init_flat, ztr)


def _mm(a, b):
    return jnp.dot(a, b, preferred_element_type=jnp.float32)


def _node_enc_body(x_ref, w1_ref, b1_ref, w2_ref, b2_ref, o_ref):
    h1 = jnp.maximum(_mm(x_ref[...], w1_ref[...]) + b1_ref[...], 0.0)
    o_ref[...] = _mm(h1, w2_ref[...]) + b2_ref[...]


def _node_enc(x, p):
    n, f = x.shape
    r = 1024
    return pl.pallas_call(
        _node_enc_body,
        grid=(n // r,),
        in_specs=[
            pl.BlockSpec((r, f), lambda i: (i, 0)),
            pl.BlockSpec((f, H), lambda i: (0, 0)),
            pl.BlockSpec((1, H), lambda i: (0, 0)),
            pl.BlockSpec((H, H), lambda i: (0, 0)),
            pl.BlockSpec((1, H), lambda i: (0, 0)),
        ],
        out_specs=pl.BlockSpec((r, H), lambda i: (i, 0)),
        out_shape=jax.ShapeDtypeStruct((n, H), jnp.float32),
    )(x, p["l1"]["W"], p["l1"]["b"][None], p["l2"]["W"], p["l2"]["b"][None])


def _edge_enc_body(ea_ref, w1_ref, b1_ref, w2_ref, b2_ref, o_ref):
    h1 = jnp.maximum(_mm(ea_ref[0], w1_ref[0]) + b1_ref[0], 0.0)
    o_ref[0] = (_mm(h1, w2_ref[0]) + b2_ref[0]).astype(jnp.bfloat16)


def _edge_enc(ea_all, w1, b1, w2, b2):
    k, e, f = ea_all.shape
    be = 4000
    return pl.pallas_call(
        _edge_enc_body,
        grid=(k, e // be),
        in_specs=[
            pl.BlockSpec((1, be, f), lambda k_, i: (k_, i, 0)),
            pl.BlockSpec((1, f, H), lambda k_, i: (k_, 0, 0)),
            pl.BlockSpec((1, 1, H), lambda k_, i: (k_, 0, 0)),
            pl.BlockSpec((1, H, H), lambda k_, i: (k_, 0, 0)),
            pl.BlockSpec((1, 1, H), lambda k_, i: (k_, 0, 0)),
        ],
        out_specs=pl.BlockSpec((1, be, H), lambda k_, i: (k_, i, 0)),
        out_shape=jax.ShapeDtypeStruct((k, e, H), jnp.bfloat16),
    )(ea_all, w1, b1, w2, b2)


def _edge_mlp_body(hs_ref, hd_ref, e_ref, ws_ref, wd_ref, we_ref, b1_ref,
                   w2_ref, b2_ref, o_ref):
    hs = hs_ref[0].astype(jnp.bfloat16)
    hd = hd_ref[0].astype(jnp.bfloat16)
    pre = (_mm(hs, ws_ref[0]) + _mm(hd, wd_ref[0])
           + _mm(e_ref[0], we_ref[0]) + b1_ref[0])
    h1 = jnp.maximum(pre, 0.0).astype(jnp.bfloat16)
    o_ref[0] = _mm(h1, w2_ref[0]) + b2_ref[0]


def _edge_mlp(g_half, e_all, ws, wd, we, b1, w2, b2, k0):
    _, e, _ = e_all.shape
    kh = 2
    be = 4000
    h2 = 2 * H
    return pl.pallas_call(
        _edge_mlp_body,
        grid=(kh, e // be),
        in_specs=[
            pl.BlockSpec((1, be, H), lambda k_, i: (k_, i, 0)),
            pl.BlockSpec((1, be, H), lambda k_, i: (k_ + kh, i, 0)),
            pl.BlockSpec((1, be, H), lambda k_, i: (k0 + k_, i, 0)),
            pl.BlockSpec((1, H, h2), lambda k_, i: (k0 + k_, 0, 0)),
            pl.BlockSpec((1, H, h2), lambda k_, i: (k0 + k_, 0, 0)),
            pl.BlockSpec((1, H, h2), lambda k_, i: (k0 + k_, 0, 0)),
            pl.BlockSpec((1, 1, h2), lambda k_, i: (k0 + k_, 0, 0)),
            pl.BlockSpec((1, h2, H), lambda k_, i: (k0 + k_, 0, 0)),
            pl.BlockSpec((1, 1, H), lambda k_, i: (k0 + k_, 0, 0)),
        ],
        out_specs=pl.BlockSpec((1, be, H), lambda k_, i: (k_, i, 0)),
        out_shape=jax.ShapeDtypeStruct((kh, e, H), jnp.float32),
    )(g_half, g_half, e_all, ws, wd, we, b1, w2, b2)


def _node_upd_body(h_ref, agg_ref, wh_ref, wa_ref, b1_ref, w2_ref, b2_ref,
                   g_ref, bn_ref, o_ref):
    h = h_ref[...]
    pre = _mm(h, wh_ref[...]) + _mm(agg_ref[...], wa_ref[...]) + b1_ref[...]
    h1 = jnp.maximum(pre, 0.0)
    y = h + _mm(h1, w2_ref[...]) + b2_ref[...]
    mu = jnp.mean(y, axis=-1, keepdims=True)
    var = jnp.mean((y - mu) ** 2, axis=-1, keepdims=True)
    o_ref[...] = (y - mu) * jax.lax.rsqrt(var + 1e-5) * g_ref[...] + bn_ref[...]


def _node_update(h, agg, wh, wa, b1, w2, b2, g, bn):
    n = h.shape[0]
    r = 1024
    h2 = 2 * H
    return pl.pallas_call(
        _node_upd_body,
        grid=(n // r,),
        in_specs=[
            pl.BlockSpec((r, H), lambda i: (i, 0)),
            pl.BlockSpec((r, H), lambda i: (i, 0)),
            pl.BlockSpec((H, h2), lambda i: (0, 0)),
            pl.BlockSpec((H, h2), lambda i: (0, 0)),
            pl.BlockSpec((1, h2), lambda i: (0, 0)),
            pl.BlockSpec((h2, H), lambda i: (0, 0)),
            pl.BlockSpec((1, H), lambda i: (0, 0)),
            pl.BlockSpec((1, H), lambda i: (0, 0)),
            pl.BlockSpec((1, H), lambda i: (0, 0)),
        ],
        out_specs=pl.BlockSpec((r, H), lambda i: (i, 0)),
        out_shape=jax.ShapeDtypeStruct((n, H), jnp.float32),
    )(h, agg, wh, wa, b1, w2, b2, g, bn)


def _heads_body(n_real, h_ref, w1_ref, b1_ref, w2r_ref, b2_ref, np_ref, gm_ref):
    i = pl.program_id(0)
    h = h_ref[...]
    r = h.shape[0]
    h1 = jnp.maximum(_mm(h, w1_ref[...]) + b1_ref[...], 0.0)
    np_ref[...] = jnp.sum(h1 * w2r_ref[...], axis=-1, keepdims=True) + b2_ref[...]
    rid = lax.broadcasted_iota(jnp.int32, (r, 1), 0) + i * r
    bmax = jnp.max(jnp.where(rid < n_real, h, -jnp.inf), axis=0, keepdims=True)

    @pl.when(i == 0)
    def _():
        gm_ref[...] = bmax

    @pl.when(i > 0)
    def _():
        gm_ref[...] = jnp.maximum(gm_ref[...], bmax)


def _heads(h, p, n_real):
    n = h.shape[0]
    r = 1024
    hh = H // 2
    return pl.pallas_call(
        functools.partial(_heads_body, n_real),
        grid=(n // r,),
        in_specs=[
            pl.BlockSpec((r, H), lambda i: (i, 0)),
            pl.BlockSpec((H, hh), lambda i: (0, 0)),
            pl.BlockSpec((1, hh), lambda i: (0, 0)),
            pl.BlockSpec((1, hh), lambda i: (0, 0)),
            pl.BlockSpec((1, 1), lambda i: (0, 0)),
        ],
        out_specs=[
            pl.BlockSpec((r, 1), lambda i: (i, 0)),
            pl.BlockSpec((1, H), lambda i: (0, 0)),
        ],
        out_shape=[
            jax.ShapeDtypeStruct((n, 1), jnp.float32),
            jax.ShapeDtypeStruct((1, H), jnp.float32),
        ],
    )(h, p["l1"]["W"], p["l1"]["b"][None], p["l2"]["W"].T, p["l2"]["b"][None])


def _graph_head_body(g_ref, w1_ref, b1_ref, w2r_ref, b2_ref, o_ref):
    h1 = jnp.maximum(_mm(g_ref[...], w1_ref[...]) + b1_ref[...], 0.0)
    o_ref[...] = jnp.sum(h1 * w2r_ref[...], axis=-1, keepdims=True) + b2_ref[...]


def _graph_head(g, p):
    hh = H // 2
    return pl.pallas_call(
        _graph_head_body,
        grid=(1,),
        in_specs=[
            pl.BlockSpec((1, H), lambda i: (0, 0)),
            pl.BlockSpec((H, hh), lambda i: (0, 0)),
            pl.BlockSpec((1, hh), lambda i: (0, 0)),
            pl.BlockSpec((1, hh), lambda i: (0, 0)),
            pl.BlockSpec((1, 1), lambda i: (0, 0)),
        ],
        out_specs=pl.BlockSpec((1, 1), lambda i: (0, 0)),
        out_shape=jax.ShapeDtypeStruct((1, 1), jnp.float32),
    )(g, p["l1"]["W"], p["l1"]["b"][None], p["l2"]["W"].T, p["l2"]["b"][None])


def kernel(x, ei0, ei1, ei2, ei3, ea0, ea1, ea2, ea3, params):
    ei = [ei0, ei1, ei2, ei3]
    ea_all = jnp.stack([ea0, ea1, ea2, ea3])

    n = x.shape[0]
    npad = NW * NT
    x_pad = jnp.pad(x, ((0, npad - n), (0, 0)))
    h = _node_enc(x_pad, params["node_enc"])

    e = ei0.shape[1]
    src_all = jnp.stack([ee[0] for ee in ei])
    dst_all = jnp.stack([ee[1] for ee in ei])

    iota_e = jnp.arange(e, dtype=jnp.int32)
    skey = jnp.sort(dst_all * 131072 + iota_e[None], axis=-1)
    perm = skey & 131071
    dst_s = skey >> 17
    src_s = jnp.take_along_axis(src_all, perm, axis=1)
    ea_s = jnp.take_along_axis(
        jnp.reshape(ea_all, (K * e, -1)),
        jnp.reshape(perm + jnp.arange(K, dtype=jnp.int32)[:, None] * e,
                    (-1,))[:, None], axis=0)
    ea_s = jnp.reshape(ea_s, (K, e, -1))
    ld_flat = jnp.reshape(dst_s % NT, (-1,))
    targets = jnp.broadcast_to(jnp.arange(NW + 1, dtype=jnp.int32) * NT,
                               (K, NW + 1))
    bounds = jax.vmap(functools.partial(jnp.searchsorted, side="left"))(
        dst_s, targets).astype(jnp.int32)
    bounds_flat = jnp.reshape(
        jnp.pad(bounds, ((0, 0), (0, 48 - (NW + 1)))), (-1,))
    zeros_npad = jnp.zeros((npad * H,), jnp.float32)
    ztr = jnp.zeros((H,), jnp.float32)
    gidx_a = jnp.reshape(jnp.concatenate([src_s[:2], dst_s[:2]]), (-1,))
    gidx_b = jnp.reshape(jnp.concatenate([src_s[2:], dst_s[2:]]), (-1,))
    ld_a, ld_b = ld_flat[:2 * e], ld_flat[2 * e:]
    bounds_a, bounds_b = bounds_flat[:96], bounds_flat[96:]

    ew1 = jnp.stack([p["l1"]["W"] for p in params["edge_encs"]])
    eb1 = jnp.stack([p["l1"]["b"][None] for p in params["edge_encs"]])
    ew2 = jnp.stack([p["l2"]["W"] for p in params["edge_encs"]])
    eb2 = jnp.stack([p["l2"]["b"][None] for p in params["edge_encs"]])
    e_all = _edge_enc(ea_s, ew1, eb1, ew2, eb2)

    for lp in params["layers"]:
        ws = jnp.stack([p["l1"]["W"][:H]
                        for p in lp["edge_mlps"]]).astype(jnp.bfloat16)
        wd = jnp.stack([p["l1"]["W"][H:2 * H]
                        for p in lp["edge_mlps"]]).astype(jnp.bfloat16)
        we = jnp.stack([p["l1"]["W"][2 * H:]
                        for p in lp["edge_mlps"]]).astype(jnp.bfloat16)
        b1 = jnp.stack([p["l1"]["b"][None] for p in lp["edge_mlps"]])
        w2 = jnp.stack([p["l2"]["W"]
                        for p in lp["edge_mlps"]]).astype(jnp.bfloat16)
        b2 = jnp.stack([p["l2"]["b"][None] for p in lp["edge_mlps"]])

        g_a = _sc_gather(h, gidx_a, 4)
        m_a = _edge_mlp(g_a, e_all, ws, wd, we, b1, w2, b2, 0)
        g_b = _sc_gather(h, gidx_b, 4)
        m_b = _edge_mlp(g_b, e_all, ws, wd, we, b1, w2, b2, 2)
        agg_a = _sc_scatter(m_a, ld_a, bounds_a, zeros_npad, ztr)
        agg_b = _sc_scatter(m_b, ld_b, bounds_b, agg_a, ztr)
        agg_pad = jnp.reshape(agg_b, (npad, H))

        nm = lp["node_mlp"]
        h = _node_update(
            h, agg_pad,
            nm["l1"]["W"][:H], nm["l1"]["W"][H:], nm["l1"]["b"][None],
            nm["l2"]["W"], nm["l2"]["b"][None],
            lp["norm"]["g"][None], lp["norm"]["b"][None])

    node_pred, gmax = _heads(h, params["reg_head"], n)
    graph_pred = _graph_head(gmax, params["graph_head"])
    return (jnp.reshape(node_pred, (-1,))[:n], jnp.reshape(graph_pred, (1,)))


# speedup vs baseline: 2.3168x; 1.0236x over previous
import functools

import jax
import jax.numpy as jnp
from jax import lax
from jax.experimental import pallas as pl
from jax.experimental.pallas import tpu as pltpu
from jax.experimental.pallas import tpu_sc as plsc

H = 128
K = 4
NW = 32


HP = H // 2


def _sc_gather(table, idx_flat, n_lists):
    e = idx_flat.shape[0] // n_lists
    ch = 400
    nch = e // ch
    tot = n_lists * nch
    iters = (tot + NW - 1) // NW
    mesh = plsc.VectorSubcoreMesh(core_axis_name="c", subcore_axis_name="s")

    @functools.partial(
        pl.kernel,
        out_type=jax.ShapeDtypeStruct((n_lists, e, H), jnp.float32),
        mesh=mesh,
        scratch_types=[
            pltpu.VMEM((ch,), jnp.int32),
            pltpu.VMEM((ch,), jnp.int32),
            pltpu.VMEM((ch, H), jnp.float32),
            pltpu.VMEM((ch, H), jnp.float32),
            pltpu.SemaphoreType.DMA,
            pltpu.SemaphoreType.DMA,
        ],
    )
    def gather_k(h_hbm, idx_hbm, out_hbm, idx0, idx1, rows0, rows1,
                 sem0, sem1):
        wid = lax.axis_index("s") * 2 + lax.axis_index("c")
        idxs = [idx0, idx1]
        rows = [rows0, rows1]
        sems = [sem0, sem1]

        def start(t, buf):
            pltpu.sync_copy(idx_hbm.at[pl.ds(t * ch, ch)], idxs[buf])
            pltpu.async_copy(h_hbm.at[idxs[buf]], rows[buf], sems[buf])

        @pl.when(wid < tot)
        def _():
            start(wid, 0)

        def drain(t, buf):
            pltpu.make_async_copy(h_hbm, rows[buf], sems[buf]).wait()
            j = t // nch
            c = t - j * nch
            pltpu.sync_copy(rows[buf], out_hbm.at[j, pl.ds(c * ch, ch)])

        def body(ip, carry):
            for b in range(2):
                i = 2 * ip + b
                t = wid + NW * i
                tn = t + NW

                @pl.when(tn < tot)
                def _(tn=tn, b=b):
                    start(tn, (b + 1) % 2)

                @pl.when(t < tot)
                def _(t=t, b=b):
                    drain(t, b)

            return carry

        lax.fori_loop(0, (iters + 1) // 2, body, 0)

    return gather_k(table, idx_flat)


NT = 320
SCH = 256


def _sc_scatter(m_all, ld_flat, bounds_flat, init_flat, ztr):
    k4, e, _ = m_all.shape
    npad = NW * NT
    mesh = plsc.VectorSubcoreMesh(core_axis_name="c", subcore_axis_name="s")

    @functools.partial(
        pl.kernel,
        out_type=jax.ShapeDtypeStruct((npad * H,), jnp.float32),
        mesh=mesh,
        compiler_params=pltpu.CompilerParams(needs_layout_passes=False),
        scratch_types=[
            pltpu.VMEM(((NT + 1) * H,), jnp.float32),
            pltpu.VMEM((SCH, H), jnp.float32),
            pltpu.VMEM((SCH,), jnp.int32),
            pltpu.VMEM((48,), jnp.int32),
            pltpu.SemaphoreType.DMA,
        ],
    )
    def scatter_k(m_hbm, ld_hbm, b_hbm, z_hbm, ztr_hbm, out_hbm,
                  acc_v, mrow_v, ld_v, b_v, sem):
        wid = lax.axis_index("s") * 2 + lax.axis_index("c")
        pltpu.sync_copy(z_hbm.at[pl.ds(wid * NT * H, NT * H)],
                        acc_v.at[pl.ds(0, NT * H)])
        pltpu.sync_copy(ztr_hbm, acc_v.at[pl.ds(NT * H, H)])
        lanes = lax.broadcasted_iota(jnp.int32, (16,), 0)
        cols = [lanes + 16 * j for j in range(H // 16)]

        def _bound(pos):
            acc = jnp.zeros((), jnp.int32)
            for rg in range(3):
                v = b_v[pl.ds(16 * rg, 16)]
                msk = (lanes + 16 * rg == pos).astype(jnp.int32)
                acc = acc + jnp.sum(v * msk)
            return acc

        for k in range(k4):
            pltpu.sync_copy(b_hbm.at[pl.ds(k * 48, 48)], b_v)
            s = _bound(wid)
            e_ = _bound(wid + 1)
            b0 = (s // 8) * 8
            nch = (e_ - b0 + SCH - 1) // SCH

            def chunk_body(i, carry, k=k, s=s, e_=e_, b0=b0):
                g0 = b0 + i * SCH
                g0c = jnp.minimum(g0, e - SCH)
                pltpu.sync_copy(m_hbm.at[k, pl.ds(g0c, SCH)], mrow_v)
                pltpu.sync_copy(ld_hbm.at[pl.ds(k * e + g0c, SCH)], ld_v)
                lo = jnp.maximum(g0, s)

                def grp_body(b, carry2):
                    lvec = ld_v[pl.ds(16 * b, 16)]
                    gbase = g0c + 16 * b
                    for rr in range(16):
                        valid = (gbase + rr >= lo) & (gbase + rr < e_)
                        dsc = lax.gather(
                            lvec, jnp.full((16, 1), rr, jnp.int32),
                            lax.GatherDimensionNumbers(
                                offset_dims=(), collapsed_slice_dims=(0,),
                                start_index_map=(0,)),
                            slice_sizes=(1,),
                            mode=lax.GatherScatterMode.PROMISE_IN_BOUNDS)
                        dvec = jnp.where(valid, dsc, NT) * H
                        for j in range(H // 16):
                            plsc.addupdate_scatter(
                                acc_v, [dvec + cols[j]],
                                mrow_v[16 * b + rr, pl.ds(16 * j, 16)])
                    return carry2

                lax.fori_loop(0, SCH // 16, grp_body, 0)
                return carry

            lax.fori_loop(0, nch, chunk_body, 0)

        pltpu.sync_copy(acc_v.at[pl.ds(0, NT * H)],
                        out_hbm.at[pl.ds(wid * NT * H, NT * H)])

    return scatter_k(m_all, ld_flat, bounds_flat, init_flat, ztr)


def _mm(a, b):
    return jnp.dot(a, b, preferred_element_type=jnp.float32)


def _node_enc_body(x_ref, w1_ref, b1_ref, w2_ref, b2_ref, o_ref):
    h1 = jnp.maximum(_mm(x_ref[...], w1_ref[...]) + b1_ref[...], 0.0)
    o_ref[...] = _mm(h1, w2_ref[...]) + b2_ref[...]


def _node_enc(x, p):
    n, f = x.shape
    r = 1024
    return pl.pallas_call(
        _node_enc_body,
        grid=(n // r,),
        in_specs=[
            pl.BlockSpec((r, f), lambda i: (i, 0)),
            pl.BlockSpec((f, H), lambda i: (0, 0)),
            pl.BlockSpec((1, H), lambda i: (0, 0)),
            pl.BlockSpec((H, H), lambda i: (0, 0)),
            pl.BlockSpec((1, H), lambda i: (0, 0)),
        ],
        out_specs=pl.BlockSpec((r, H), lambda i: (i, 0)),
        out_shape=jax.ShapeDtypeStruct((n, H), jnp.float32),
    )(x, p["l1"]["W"], p["l1"]["b"][None], p["l2"]["W"], p["l2"]["b"][None])


def _edge_enc_body(ea_ref, w1_ref, b1_ref, w2_ref, b2_ref, o_ref):
    h1 = jnp.maximum(_mm(ea_ref[0], w1_ref[0]) + b1_ref[0], 0.0)
    o_ref[0] = (_mm(h1, w2_ref[0]) + b2_ref[0]).astype(jnp.bfloat16)


def _edge_enc(ea_all, w1, b1, w2, b2):
    k, e, f = ea_all.shape
    be = 4000
    return pl.pallas_call(
        _edge_enc_body,
        grid=(k, e // be),
        in_specs=[
            pl.BlockSpec((1, be, f), lambda k_, i: (k_, i, 0)),
            pl.BlockSpec((1, f, H), lambda k_, i: (k_, 0, 0)),
            pl.BlockSpec((1, 1, H), lambda k_, i: (k_, 0, 0)),
            pl.BlockSpec((1, H, H), lambda k_, i: (k_, 0, 0)),
            pl.BlockSpec((1, 1, H), lambda k_, i: (k_, 0, 0)),
        ],
        out_specs=pl.BlockSpec((1, be, H), lambda k_, i: (k_, i, 0)),
        out_shape=jax.ShapeDtypeStruct((k, e, H), jnp.bfloat16),
    )(ea_all, w1, b1, w2, b2)


def _edge_mlp_body(hs_ref, hd_ref, e_ref, ws_ref, wd_ref, we_ref, b1_ref,
                   w2_ref, b2_ref, o_ref):
    hs = hs_ref[0].astype(jnp.bfloat16)
    hd = hd_ref[0].astype(jnp.bfloat16)
    pre = (_mm(hs, ws_ref[0]) + _mm(hd, wd_ref[0])
           + _mm(e_ref[0], we_ref[0]) + b1_ref[0])
    h1 = jnp.maximum(pre, 0.0).astype(jnp.bfloat16)
    o_ref[0] = _mm(h1, w2_ref[0]) + b2_ref[0]


def _edge_mlp(g_half, e_all, ws, wd, we, b1, w2, b2, k0):
    _, e, _ = e_all.shape
    kh = 2
    be = 4000
    h2 = 2 * H
    return pl.pallas_call(
        _edge_mlp_body,
        grid=(kh, e // be),
        in_specs=[
            pl.BlockSpec((1, be, H), lambda k_, i: (k_, i, 0)),
            pl.BlockSpec((1, be, H), lambda k_, i: (k_ + kh, i, 0)),
            pl.BlockSpec((1, be, H), lambda k_, i: (k0 + k_, i, 0)),
            pl.BlockSpec((1, H, h2), lambda k_, i: (k0 + k_, 0, 0)),
            pl.BlockSpec((1, H, h2), lambda k_, i: (k0 + k_, 0, 0)),
            pl.BlockSpec((1, H, h2), lambda k_, i: (k0 + k_, 0, 0)),
            pl.BlockSpec((1, 1, h2), lambda k_, i: (k0 + k_, 0, 0)),
            pl.BlockSpec((1, h2, H), lambda k_, i: (k0 + k_, 0, 0)),
            pl.BlockSpec((1, 1, H), lambda k_, i: (k0 + k_, 0, 0)),
        ],
        out_specs=pl.BlockSpec((1, be, H), lambda k_, i: (k_, i, 0)),
        out_shape=jax.ShapeDtypeStruct((kh, e, H), jnp.float32),
    )(g_half, g_half, e_all, ws, wd, we, b1, w2, b2)


def _node_upd_body(h_ref, agg_ref, wh_ref, wa_ref, b1_ref, w2_ref, b2_ref,
                   g_ref, bn_ref, o_ref):
    h = h_ref[...]
    pre = _mm(h, wh_ref[...]) + _mm(agg_ref[...], wa_ref[...]) + b1_ref[...]
    h1 = jnp.maximum(pre, 0.0)
    y = h + _mm(h1, w2_ref[...]) + b2_ref[...]
    mu = jnp.mean(y, axis=-1, keepdims=True)
    var = jnp.mean((y - mu) ** 2, axis=-1, keepdims=True)
    o_ref[...] = (y - mu) * jax.lax.rsqrt(var + 1e-5) * g_ref[...] + bn_ref[...]


def _node_update(h, agg, wh, wa, b1, w2, b2, g, bn):
    n = h.shape[0]
    r = 1024
    h2 = 2 * H
    return pl.pallas_call(
        _node_upd_body,
        grid=(n // r,),
        in_specs=[
            pl.BlockSpec((r, H), lambda i: (i, 0)),
            pl.BlockSpec((r, H), lambda i: (i, 0)),
            pl.BlockSpec((H, h2), lambda i: (0, 0)),
            pl.BlockSpec((H, h2), lambda i: (0, 0)),
            pl.BlockSpec((1, h2), lambda i: (0, 0)),
            pl.BlockSpec((h2, H), lambda i: (0, 0)),
            pl.BlockSpec((1, H), lambda i: (0, 0)),
            pl.BlockSpec((1, H), lambda i: (0, 0)),
            pl.BlockSpec((1, H), lambda i: (0, 0)),
        ],
        out_specs=pl.BlockSpec((r, H), lambda i: (i, 0)),
        out_shape=jax.ShapeDtypeStruct((n, H), jnp.float32),
    )(h, agg, wh, wa, b1, w2, b2, g, bn)


def _heads_body(n_real, h_ref, w1_ref, b1_ref, w2r_ref, b2_ref, np_ref, gm_ref):
    i = pl.program_id(0)
    h = h_ref[...]
    r = h.shape[0]
    h1 = jnp.maximum(_mm(h, w1_ref[...]) + b1_ref[...], 0.0)
    np_ref[...] = jnp.sum(h1 * w2r_ref[...], axis=-1, keepdims=True) + b2_ref[...]
    rid = lax.broadcasted_iota(jnp.int32, (r, 1), 0) + i * r
    bmax = jnp.max(jnp.where(rid < n_real, h, -jnp.inf), axis=0, keepdims=True)

    @pl.when(i == 0)
    def _():
        gm_ref[...] = bmax

    @pl.when(i > 0)
    def _():
        gm_ref[...] = jnp.maximum(gm_ref[...], bmax)


def _heads(h, p, n_real):
    n = h.shape[0]
    r = 1024
    hh = H // 2
    return pl.pallas_call(
        functools.partial(_heads_body, n_real),
        grid=(n // r,),
        in_specs=[
            pl.BlockSpec((r, H), lambda i: (i, 0)),
            pl.BlockSpec((H, hh), lambda i: (0, 0)),
            pl.BlockSpec((1, hh), lambda i: (0, 0)),
            pl.BlockSpec((1, hh), lambda i: (0, 0)),
            pl.BlockSpec((1, 1), lambda i: (0, 0)),
        ],
        out_specs=[
            pl.BlockSpec((r, 1), lambda i: (i, 0)),
            pl.BlockSpec((1, H), lambda i: (0, 0)),
        ],
        out_shape=[
            jax.ShapeDtypeStruct((n, 1), jnp.float32),
            jax.ShapeDtypeStruct((1, H), jnp.float32),
        ],
    )(h, p["l1"]["W"], p["l1"]["b"][None], p["l2"]["W"].T, p["l2"]["b"][None])


def _graph_head_body(g_ref, w1_ref, b1_ref, w2r_ref, b2_ref, o_ref):
    h1 = jnp.maximum(_mm(g_ref[...], w1_ref[...]) + b1_ref[...], 0.0)
    o_ref[...] = jnp.sum(h1 * w2r_ref[...], axis=-1, keepdims=True) + b2_ref[...]


def _graph_head(g, p):
    hh = H // 2
    return pl.pallas_call(
        _graph_head_body,
        grid=(1,),
        in_specs=[
            pl.BlockSpec((1, H), lambda i: (0, 0)),
            pl.BlockSpec((H, hh), lambda i: (0, 0)),
            pl.BlockSpec((1, hh), lambda i: (0, 0)),
            pl.BlockSpec((1, hh), lambda i: (0, 0)),
            pl.BlockSpec((1, 1), lambda i: (0, 0)),
        ],
        out_specs=pl.BlockSpec((1, 1), lambda i: (0, 0)),
        out_shape=jax.ShapeDtypeStruct((1, 1), jnp.float32),
    )(g, p["l1"]["W"], p["l1"]["b"][None], p["l2"]["W"].T, p["l2"]["b"][None])


def kernel(x, ei0, ei1, ei2, ei3, ea0, ea1, ea2, ea3, params):
    ei = [ei0, ei1, ei2, ei3]
    ea_all = jnp.stack([ea0, ea1, ea2, ea3])

    n = x.shape[0]
    npad = NW * NT
    x_pad = jnp.pad(x, ((0, npad - n), (0, 0)))
    h = _node_enc(x_pad, params["node_enc"])

    e = ei0.shape[1]
    src_all = jnp.stack([ee[0] for ee in ei])
    dst_all = jnp.stack([ee[1] for ee in ei])

    iota_e = jnp.arange(e, dtype=jnp.int32)
    skey = jnp.sort(dst_all * 131072 + iota_e[None], axis=-1)
    perm = skey & 131071
    dst_s = skey >> 17
    src_s = jnp.take_along_axis(src_all, perm, axis=1)
    ea_s = jnp.take_along_axis(
        jnp.reshape(ea_all, (K * e, -1)),
        jnp.reshape(perm + jnp.arange(K, dtype=jnp.int32)[:, None] * e,
                    (-1,))[:, None], axis=0)
    ea_s = jnp.reshape(ea_s, (K, e, -1))
    ld_flat = jnp.reshape(dst_s % NT, (-1,))
    targets = jnp.broadcast_to(jnp.arange(NW + 1, dtype=jnp.int32) * NT,
                               (K, NW + 1))
    bounds = jax.vmap(functools.partial(jnp.searchsorted, side="left"))(
        dst_s, targets).astype(jnp.int32)
    bounds_flat = jnp.reshape(
        jnp.pad(bounds, ((0, 0), (0, 48 - (NW + 1)))), (-1,))
    zeros_npad = jnp.zeros((npad * H,), jnp.float32)
    ztr = jnp.zeros((H,), jnp.float32)
    gidx_a = jnp.reshape(jnp.concatenate([src_s[:2], dst_s[:2]]), (-1,))
    gidx_b = jnp.reshape(jnp.concatenate([src_s[2:], dst_s[2:]]), (-1,))
    ld_a, ld_b = ld_flat[:2 * e], ld_flat[2 * e:]
    bounds_a, bounds_b = bounds_flat[:96], bounds_flat[96:]

    ew1 = jnp.stack([p["l1"]["W"] for p in params["edge_encs"]])
    eb1 = jnp.stack([p["l1"]["b"][None] for p in params["edge_encs"]])
    ew2 = jnp.stack([p["l2"]["W"] for p in params["edge_encs"]])
    eb2 = jnp.stack([p["l2"]["b"][None] for p in params["edge_encs"]])
    e_all = _edge_enc(ea_s, ew1, eb1, ew2, eb2)

    for lp in params["layers"]:
        ws = jnp.stack([p["l1"]["W"][:H]
                        for p in lp["edge_mlps"]]).astype(jnp.bfloat16)
        wd = jnp.stack([p["l1"]["W"][H:2 * H]
                        for p in lp["edge_mlps"]]).astype(jnp.bfloat16)
        we = jnp.stack([p["l1"]["W"][2 * H:]
                        for p in lp["edge_mlps"]]).astype(jnp.bfloat16)
        b1 = jnp.stack([p["l1"]["b"][None] for p in lp["edge_mlps"]])
        w2 = jnp.stack([p["l2"]["W"]
                        for p in lp["edge_mlps"]]).astype(jnp.bfloat16)
        b2 = jnp.stack([p["l2"]["b"][None] for p in lp["edge_mlps"]])

        g_a = _sc_gather(h, gidx_a, 4)
        m_a = _edge_mlp(g_a, e_all, ws, wd, we, b1, w2, b2, 0)
        g_b = _sc_gather(h, gidx_b, 4)
        m_b = _edge_mlp(g_b, e_all, ws, wd, we, b1, w2, b2, 2)
        agg_a = _sc_scatter(m_a, ld_a, bounds_a, zeros_npad, ztr)
        agg_b = _sc_scatter(m_b, ld_b, bounds_b, agg_a, ztr)
        agg_pad = jnp.reshape(agg_b, (npad, H))

        nm = lp["node_mlp"]
        h = _node_update(
            h, agg_pad,
            nm["l1"]["W"][:H], nm["l1"]["W"][H:], nm["l1"]["b"][None],
            nm["l2"]["W"], nm["l2"]["b"][None],
            lp["norm"]["g"][None], lp["norm"]["b"][None])

    node_pred, gmax = _heads(h, params["reg_head"], n)
    graph_pred = _graph_head(gmax, params["graph_head"])
    return (jnp.reshape(node_pred, (-1,))[:n], jnp.reshape(graph_pred, (1,)))


# speedup vs baseline: 2.5458x; 1.0988x over previous
import functools

import jax
import jax.numpy as jnp
from jax import lax
from jax.experimental import pallas as pl
from jax.experimental.pallas import tpu as pltpu
from jax.experimental.pallas import tpu_sc as plsc

H = 128
K = 4
NW = 32


HP = H // 2


def _sc_gather(table, idx_flat, n_lists):
    e = idx_flat.shape[0] // n_lists
    ch = 400
    nch = e // ch
    tot = n_lists * nch
    iters = (tot + NW - 1) // NW
    mesh = plsc.VectorSubcoreMesh(core_axis_name="c", subcore_axis_name="s")

    @functools.partial(
        pl.kernel,
        out_type=jax.ShapeDtypeStruct((n_lists, e, H), jnp.float32),
        mesh=mesh,
        scratch_types=[
            pltpu.VMEM((ch,), jnp.int32),
            pltpu.VMEM((ch,), jnp.int32),
            pltpu.VMEM((ch, H), jnp.float32),
            pltpu.VMEM((ch, H), jnp.float32),
            pltpu.SemaphoreType.DMA,
            pltpu.SemaphoreType.DMA,
        ],
    )
    def gather_k(h_hbm, idx_hbm, out_hbm, idx0, idx1, rows0, rows1,
                 sem0, sem1):
        wid = lax.axis_index("s") * 2 + lax.axis_index("c")
        idxs = [idx0, idx1]
        rows = [rows0, rows1]
        sems = [sem0, sem1]

        def start(t, buf):
            pltpu.sync_copy(idx_hbm.at[pl.ds(t * ch, ch)], idxs[buf])
            pltpu.async_copy(h_hbm.at[idxs[buf]], rows[buf], sems[buf])

        @pl.when(wid < tot)
        def _():
            start(wid, 0)

        def drain(t, buf):
            pltpu.make_async_copy(h_hbm, rows[buf], sems[buf]).wait()
            j = t // nch
            c = t - j * nch
            pltpu.sync_copy(rows[buf], out_hbm.at[j, pl.ds(c * ch, ch)])

        def body(ip, carry):
            for b in range(2):
                i = 2 * ip + b
                t = wid + NW * i
                tn = t + NW

                @pl.when(tn < tot)
                def _(tn=tn, b=b):
                    start(tn, (b + 1) % 2)

                @pl.when(t < tot)
                def _(t=t, b=b):
                    drain(t, b)

            return carry

        lax.fori_loop(0, (iters + 1) // 2, body, 0)

    return gather_k(table, idx_flat)


NT = 320
SCH = 256


def _sc_scatter(m_all, ld_flat, bounds_flat, init_flat, ztr):
    k4, e, _ = m_all.shape
    npad = NW * NT
    mesh = plsc.VectorSubcoreMesh(core_axis_name="c", subcore_axis_name="s")

    @functools.partial(
        pl.kernel,
        out_type=jax.ShapeDtypeStruct((npad * H,), jnp.float32),
        mesh=mesh,
        compiler_params=pltpu.CompilerParams(needs_layout_passes=False),
        scratch_types=[
            pltpu.VMEM(((NT + 1) * H,), jnp.float32),
            pltpu.VMEM((SCH, H), jnp.float32),
            pltpu.VMEM((SCH, H), jnp.float32),
            pltpu.VMEM((SCH,), jnp.int32),
            pltpu.VMEM((SCH,), jnp.int32),
            pltpu.VMEM((48,), jnp.int32),
            pltpu.SemaphoreType.DMA,
            pltpu.SemaphoreType.DMA,
        ],
    )
    def scatter_k(m_hbm, ld_hbm, b_hbm, z_hbm, ztr_hbm, out_hbm,
                  acc_v, mrow0, mrow1, ld0, ld1, b_v, sm0, sm1):
        wid = lax.axis_index("s") * 2 + lax.axis_index("c")
        pltpu.sync_copy(z_hbm.at[pl.ds(wid * NT * H, NT * H)],
                        acc_v.at[pl.ds(0, NT * H)])
        pltpu.sync_copy(ztr_hbm, acc_v.at[pl.ds(NT * H, H)])
        lanes = lax.broadcasted_iota(jnp.int32, (16,), 0)
        cols = [lanes + 16 * j for j in range(H // 16)]
        mrows = [mrow0, mrow1]
        lds = [ld0, ld1]
        sems = [sm0, sm1]

        def _bound(pos):
            acc = jnp.zeros((), jnp.int32)
            for rg in range(3):
                v = b_v[pl.ds(16 * rg, 16)]
                msk = (lanes + 16 * rg == pos).astype(jnp.int32)
                acc = acc + jnp.sum(v * msk)
            return acc

        for k in range(k4):
            pltpu.sync_copy(b_hbm.at[pl.ds(k * 48, 48)], b_v)
            s = _bound(wid)
            e_ = _bound(wid + 1)
            b0 = (s // 8) * 8
            nch = (e_ - b0 + SCH - 1) // SCH

            def startc(i, buf, k=k, b0=b0):
                g0c = jnp.minimum(b0 + i * SCH, e - SCH)
                pltpu.async_copy(m_hbm.at[k, pl.ds(g0c, SCH)],
                                 mrows[buf], sems[buf])
                pltpu.async_copy(ld_hbm.at[pl.ds(k * e + g0c, SCH)],
                                 lds[buf], sems[buf])

            def compute(i, buf, s=s, e_=e_, b0=b0):
                pltpu.make_async_copy(m_hbm, mrows[buf], sems[buf]).wait()
                pltpu.make_async_copy(ld_hbm, lds[buf], sems[buf]).wait()
                g0 = b0 + i * SCH
                g0c = jnp.minimum(g0, e - SCH)
                lo = jnp.maximum(g0, s)
                mrow_v = mrows[buf]
                ld_v = lds[buf]

                def grp_body(b, carry2):
                    lvec = ld_v[pl.ds(16 * b, 16)]
                    gbase = g0c + 16 * b
                    for rr in range(16):
                        valid = (gbase + rr >= lo) & (gbase + rr < e_)
                        dsc = lax.gather(
                            lvec, jnp.full((16, 1), rr, jnp.int32),
                            lax.GatherDimensionNumbers(
                                offset_dims=(), collapsed_slice_dims=(0,),
                                start_index_map=(0,)),
                            slice_sizes=(1,),
                            mode=lax.GatherScatterMode.PROMISE_IN_BOUNDS)
                        dvec = jnp.where(valid, dsc, NT) * H
                        for j in range(H // 16):
                            plsc.addupdate_scatter(
                                acc_v, [dvec + cols[j]],
                                mrow_v[16 * b + rr, pl.ds(16 * j, 16)])
                    return carry2

                lax.fori_loop(0, SCH // 16, grp_body, 0)

            @pl.when(nch > 0)
            def _(startc=startc):
                startc(0, 0)

            def chunk_pair(ip, carry, nch=nch, startc=startc, compute=compute):
                for b in range(2):
                    i = 2 * ip + b

                    @pl.when(i + 1 < nch)
                    def _(i=i, b=b):
                        startc(i + 1, (b + 1) % 2)

                    @pl.when(i < nch)
                    def _(i=i, b=b):
                        compute(i, b)

                return carry

            lax.fori_loop(0, (nch + 1) // 2, chunk_pair, 0)

        pltpu.sync_copy(acc_v.at[pl.ds(0, NT * H)],
                        out_hbm.at[pl.ds(wid * NT * H, NT * H)])

    return scatter_k(m_all, ld_flat, bounds_flat, init_flat, ztr)


def _mm(a, b):
    return jnp.dot(a, b, preferred_element_type=jnp.float32)


def _node_enc_body(x_ref, w1_ref, b1_ref, w2_ref, b2_ref, o_ref):
    h1 = jnp.maximum(_mm(x_ref[...], w1_ref[...]) + b1_ref[...], 0.0)
    o_ref[...] = _mm(h1, w2_ref[...]) + b2_ref[...]


def _node_enc(x, p):
    n, f = x.shape
    r = 1024
    return pl.pallas_call(
        _node_enc_body,
        grid=(n // r,),
        in_specs=[
            pl.BlockSpec((r, f), lambda i: (i, 0)),
            pl.BlockSpec((f, H), lambda i: (0, 0)),
            pl.BlockSpec((1, H), lambda i: (0, 0)),
            pl.BlockSpec((H, H), lambda i: (0, 0)),
            pl.BlockSpec((1, H), lambda i: (0, 0)),
        ],
        out_specs=pl.BlockSpec((r, H), lambda i: (i, 0)),
        out_shape=jax.ShapeDtypeStruct((n, H), jnp.float32),
    )(x, p["l1"]["W"], p["l1"]["b"][None], p["l2"]["W"], p["l2"]["b"][None])


def _edge_enc_body(ea_ref, w1_ref, b1_ref, w2_ref, b2_ref, o_ref):
    h1 = jnp.maximum(_mm(ea_ref[0], w1_ref[0]) + b1_ref[0], 0.0)
    o_ref[0] = (_mm(h1, w2_ref[0]) + b2_ref[0]).astype(jnp.bfloat16)


def _edge_enc(ea_all, w1, b1, w2, b2):
    k, e, f = ea_all.shape
    be = 4000
    return pl.pallas_call(
        _edge_enc_body,
        grid=(k, e // be),
        in_specs=[
            pl.BlockSpec((1, be, f), lambda k_, i: (k_, i, 0)),
            pl.BlockSpec((1, f, H), lambda k_, i: (k_, 0, 0)),
            pl.BlockSpec((1, 1, H), lambda k_, i: (k_, 0, 0)),
            pl.BlockSpec((1, H, H), lambda k_, i: (k_, 0, 0)),
            pl.BlockSpec((1, 1, H), lambda k_, i: (k_, 0, 0)),
        ],
        out_specs=pl.BlockSpec((1, be, H), lambda k_, i: (k_, i, 0)),
        out_shape=jax.ShapeDtypeStruct((k, e, H), jnp.bfloat16),
    )(ea_all, w1, b1, w2, b2)


def _edge_mlp_body(hs_ref, hd_ref, e_ref, ws_ref, wd_ref, we_ref, b1_ref,
                   w2_ref, b2_ref, o_ref):
    hs = hs_ref[0].astype(jnp.bfloat16)
    hd = hd_ref[0].astype(jnp.bfloat16)
    pre = (_mm(hs, ws_ref[0]) + _mm(hd, wd_ref[0])
           + _mm(e_ref[0], we_ref[0]) + b1_ref[0])
    h1 = jnp.maximum(pre, 0.0).astype(jnp.bfloat16)
    o_ref[0] = _mm(h1, w2_ref[0]) + b2_ref[0]


def _edge_mlp(g_half, e_all, ws, wd, we, b1, w2, b2, k0):
    _, e, _ = e_all.shape
    kh = 2
    be = 4000
    h2 = 2 * H
    return pl.pallas_call(
        _edge_mlp_body,
        grid=(kh, e // be),
        in_specs=[
            pl.BlockSpec((1, be, H), lambda k_, i: (k_, i, 0)),
            pl.BlockSpec((1, be, H), lambda k_, i: (k_ + kh, i, 0)),
            pl.BlockSpec((1, be, H), lambda k_, i: (k0 + k_, i, 0)),
            pl.BlockSpec((1, H, h2), lambda k_, i: (k0 + k_, 0, 0)),
            pl.BlockSpec((1, H, h2), lambda k_, i: (k0 + k_, 0, 0)),
            pl.BlockSpec((1, H, h2), lambda k_, i: (k0 + k_, 0, 0)),
            pl.BlockSpec((1, 1, h2), lambda k_, i: (k0 + k_, 0, 0)),
            pl.BlockSpec((1, h2, H), lambda k_, i: (k0 + k_, 0, 0)),
            pl.BlockSpec((1, 1, H), lambda k_, i: (k0 + k_, 0, 0)),
        ],
        out_specs=pl.BlockSpec((1, be, H), lambda k_, i: (k_, i, 0)),
        out_shape=jax.ShapeDtypeStruct((kh, e, H), jnp.float32),
    )(g_half, g_half, e_all, ws, wd, we, b1, w2, b2)


def _node_upd_body(h_ref, agg_ref, wh_ref, wa_ref, b1_ref, w2_ref, b2_ref,
                   g_ref, bn_ref, o_ref):
    h = h_ref[...]
    pre = _mm(h, wh_ref[...]) + _mm(agg_ref[...], wa_ref[...]) + b1_ref[...]
    h1 = jnp.maximum(pre, 0.0)
    y = h + _mm(h1, w2_ref[...]) + b2_ref[...]
    mu = jnp.mean(y, axis=-1, keepdims=True)
    var = jnp.mean((y - mu) ** 2, axis=-1, keepdims=True)
    o_ref[...] = (y - mu) * jax.lax.rsqrt(var + 1e-5) * g_ref[...] + bn_ref[...]


def _node_update(h, agg, wh, wa, b1, w2, b2, g, bn):
    n = h.shape[0]
    r = 1024
    h2 = 2 * H
    return pl.pallas_call(
        _node_upd_body,
        grid=(n // r,),
        in_specs=[
            pl.BlockSpec((r, H), lambda i: (i, 0)),
            pl.BlockSpec((r, H), lambda i: (i, 0)),
            pl.BlockSpec((H, h2), lambda i: (0, 0)),
            pl.BlockSpec((H, h2), lambda i: (0, 0)),
            pl.BlockSpec((1, h2), lambda i: (0, 0)),
            pl.BlockSpec((h2, H), lambda i: (0, 0)),
            pl.BlockSpec((1, H), lambda i: (0, 0)),
            pl.BlockSpec((1, H), lambda i: (0, 0)),
            pl.BlockSpec((1, H), lambda i: (0, 0)),
        ],
        out_specs=pl.BlockSpec((r, H), lambda i: (i, 0)),
        out_shape=jax.ShapeDtypeStruct((n, H), jnp.float32),
    )(h, agg, wh, wa, b1, w2, b2, g, bn)


def _heads_body(n_real, h_ref, w1_ref, b1_ref, w2r_ref, b2_ref, np_ref, gm_ref):
    i = pl.program_id(0)
    h = h_ref[...]
    r = h.shape[0]
    h1 = jnp.maximum(_mm(h, w1_ref[...]) + b1_ref[...], 0.0)
    np_ref[...] = jnp.sum(h1 * w2r_ref[...], axis=-1, keepdims=True) + b2_ref[...]
    rid = lax.broadcasted_iota(jnp.int32, (r, 1), 0) + i * r
    bmax = jnp.max(jnp.where(rid < n_real, h, -jnp.inf), axis=0, keepdims=True)

    @pl.when(i == 0)
    def _():
        gm_ref[...] = bmax

    @pl.when(i > 0)
    def _():
        gm_ref[...] = jnp.maximum(gm_ref[...], bmax)


def _heads(h, p, n_real):
    n = h.shape[0]
    r = 1024
    hh = H // 2
    return pl.pallas_call(
        functools.partial(_heads_body, n_real),
        grid=(n // r,),
        in_specs=[
            pl.BlockSpec((r, H), lambda i: (i, 0)),
            pl.BlockSpec((H, hh), lambda i: (0, 0)),
            pl.BlockSpec((1, hh), lambda i: (0, 0)),
            pl.BlockSpec((1, hh), lambda i: (0, 0)),
            pl.BlockSpec((1, 1), lambda i: (0, 0)),
        ],
        out_specs=[
            pl.BlockSpec((r, 1), lambda i: (i, 0)),
            pl.BlockSpec((1, H), lambda i: (0, 0)),
        ],
        out_shape=[
            jax.ShapeDtypeStruct((n, 1), jnp.float32),
            jax.ShapeDtypeStruct((1, H), jnp.float32),
        ],
    )(h, p["l1"]["W"], p["l1"]["b"][None], p["l2"]["W"].T, p["l2"]["b"][None])


def _graph_head_body(g_ref, w1_ref, b1_ref, w2r_ref, b2_ref, o_ref):
    h1 = jnp.maximum(_mm(g_ref[...], w1_ref[...]) + b1_ref[...], 0.0)
    o_ref[...] = jnp.sum(h1 * w2r_ref[...], axis=-1, keepdims=True) + b2_ref[...]


def _graph_head(g, p):
    hh = H // 2
    return pl.pallas_call(
        _graph_head_body,
        grid=(1,),
        in_specs=[
            pl.BlockSpec((1, H), lambda i: (0, 0)),
            pl.BlockSpec((H, hh), lambda i: (0, 0)),
            pl.BlockSpec((1, hh), lambda i: (0, 0)),
            pl.BlockSpec((1, hh), lambda i: (0, 0)),
            pl.BlockSpec((1, 1), lambda i: (0, 0)),
        ],
        out_specs=pl.BlockSpec((1, 1), lambda i: (0, 0)),
        out_shape=jax.ShapeDtypeStruct((1, 1), jnp.float32),
    )(g, p["l1"]["W"], p["l1"]["b"][None], p["l2"]["W"].T, p["l2"]["b"][None])


def kernel(x, ei0, ei1, ei2, ei3, ea0, ea1, ea2, ea3, params):
    ei = [ei0, ei1, ei2, ei3]
    ea_all = jnp.stack([ea0, ea1, ea2, ea3])

    n = x.shape[0]
    npad = NW * NT
    x_pad = jnp.pad(x, ((0, npad - n), (0, 0)))
    h = _node_enc(x_pad, params["node_enc"])

    e = ei0.shape[1]
    src_all = jnp.stack([ee[0] for ee in ei])
    dst_all = jnp.stack([ee[1] for ee in ei])

    iota_e = jnp.arange(e, dtype=jnp.int32)
    skey = jnp.sort(dst_all * 131072 + iota_e[None], axis=-1)
    perm = skey & 131071
    dst_s = skey >> 17
    src_s = jnp.take_along_axis(src_all, perm, axis=1)
    ea_s = jnp.take_along_axis(
        jnp.reshape(ea_all, (K * e, -1)),
        jnp.reshape(perm + jnp.arange(K, dtype=jnp.int32)[:, None] * e,
                    (-1,))[:, None], axis=0)
    ea_s = jnp.reshape(ea_s, (K, e, -1))
    ld_flat = jnp.reshape(dst_s % NT, (-1,))
    targets = jnp.broadcast_to(jnp.arange(NW + 1, dtype=jnp.int32) * NT,
                               (K, NW + 1))
    bounds = jax.vmap(functools.partial(jnp.searchsorted, side="left"))(
        dst_s, targets).astype(jnp.int32)
    bounds_flat = jnp.reshape(
        jnp.pad(bounds, ((0, 0), (0, 48 - (NW + 1)))), (-1,))
    zeros_npad = jnp.zeros((npad * H,), jnp.float32)
    ztr = jnp.zeros((H,), jnp.float32)
    gidx_a = jnp.reshape(jnp.concatenate([src_s[:2], dst_s[:2]]), (-1,))
    gidx_b = jnp.reshape(jnp.concatenate([src_s[2:], dst_s[2:]]), (-1,))
    ld_a, ld_b = ld_flat[:2 * e], ld_flat[2 * e:]
    bounds_a, bounds_b = bounds_flat[:96], bounds_flat[96:]

    ew1 = jnp.stack([p["l1"]["W"] for p in params["edge_encs"]])
    eb1 = jnp.stack([p["l1"]["b"][None] for p in params["edge_encs"]])
    ew2 = jnp.stack([p["l2"]["W"] for p in params["edge_encs"]])
    eb2 = jnp.stack([p["l2"]["b"][None] for p in params["edge_encs"]])
    e_all = _edge_enc(ea_s, ew1, eb1, ew2, eb2)

    for lp in params["layers"]:
        ws = jnp.stack([p["l1"]["W"][:H]
                        for p in lp["edge_mlps"]]).astype(jnp.bfloat16)
        wd = jnp.stack([p["l1"]["W"][H:2 * H]
                        for p in lp["edge_mlps"]]).astype(jnp.bfloat16)
        we = jnp.stack([p["l1"]["W"][2 * H:]
                        for p in lp["edge_mlps"]]).astype(jnp.bfloat16)
        b1 = jnp.stack([p["l1"]["b"][None] for p in lp["edge_mlps"]])
        w2 = jnp.stack([p["l2"]["W"]
                        for p in lp["edge_mlps"]]).astype(jnp.bfloat16)
        b2 = jnp.stack([p["l2"]["b"][None] for p in lp["edge_mlps"]])

        g_a = _sc_gather(h, gidx_a, 4)
        m_a = _edge_mlp(g_a, e_all, ws, wd, we, b1, w2, b2, 0)
        g_b = _sc_gather(h, gidx_b, 4)
        m_b = _edge_mlp(g_b, e_all, ws, wd, we, b1, w2, b2, 2)
        agg_a = _sc_scatter(m_a, ld_a, bounds_a, zeros_npad, ztr)
        agg_b = _sc_scatter(m_b, ld_b, bounds_b, agg_a, ztr)
        agg_pad = jnp.reshape(agg_b, (npad, H))

        nm = lp["node_mlp"]
        h = _node_update(
            h, agg_pad,
            nm["l1"]["W"][:H], nm["l1"]["W"][H:], nm["l1"]["b"][None],
            nm["l2"]["W"], nm["l2"]["b"][None],
            lp["norm"]["g"][None], lp["norm"]["b"][None])

    node_pred, gmax = _heads(h, params["reg_head"], n)
    graph_pred = _graph_head(gmax, params["graph_head"])
    return (jnp.reshape(node_pred, (-1,))[:n], jnp.reshape(graph_pred, (1,)))


# speedup vs baseline: 2.8013x; 1.1004x over previous
import functools

import jax
import jax.numpy as jnp
from jax import lax
from jax.experimental import pallas as pl
from jax.experimental.pallas import tpu as pltpu
from jax.experimental.pallas import tpu_sc as plsc

H = 128
K = 4
NW = 32


HP = H // 2


def _sc_gather(table, idx_flat, n_lists):
    e = idx_flat.shape[0] // n_lists
    ch = 400
    nch = e // ch
    tot = n_lists * nch
    iters = (tot + NW - 1) // NW
    mesh = plsc.VectorSubcoreMesh(core_axis_name="c", subcore_axis_name="s")

    @functools.partial(
        pl.kernel,
        out_type=jax.ShapeDtypeStruct((n_lists, e, H), jnp.float32),
        mesh=mesh,
        scratch_types=[
            pltpu.VMEM((ch,), jnp.int32),
            pltpu.VMEM((ch,), jnp.int32),
            pltpu.VMEM((ch, H), jnp.float32),
            pltpu.VMEM((ch, H), jnp.float32),
            pltpu.SemaphoreType.DMA,
            pltpu.SemaphoreType.DMA,
        ],
    )
    def gather_k(h_hbm, idx_hbm, out_hbm, idx0, idx1, rows0, rows1,
                 sem0, sem1):
        wid = lax.axis_index("s") * 2 + lax.axis_index("c")
        idxs = [idx0, idx1]
        rows = [rows0, rows1]
        sems = [sem0, sem1]

        def start(t, buf):
            pltpu.sync_copy(idx_hbm.at[pl.ds(t * ch, ch)], idxs[buf])
            pltpu.async_copy(h_hbm.at[idxs[buf]], rows[buf], sems[buf])

        @pl.when(wid < tot)
        def _():
            start(wid, 0)

        def drain(t, buf):
            pltpu.make_async_copy(h_hbm, rows[buf], sems[buf]).wait()
            j = t // nch
            c = t - j * nch
            pltpu.sync_copy(rows[buf], out_hbm.at[j, pl.ds(c * ch, ch)])

        def body(ip, carry):
            for b in range(2):
                i = 2 * ip + b
                t = wid + NW * i
                tn = t + NW

                @pl.when(tn < tot)
                def _(tn=tn, b=b):
                    start(tn, (b + 1) % 2)

                @pl.when(t < tot)
                def _(t=t, b=b):
                    drain(t, b)

            return carry

        lax.fori_loop(0, (iters + 1) // 2, body, 0)

    return gather_k(table, idx_flat)


NT = 320
SCH = 256


def _sc_scatter(m_all, ld_flat, bounds_flat, init_flat, ztr):
    k4, e, _ = m_all.shape
    npad = NW * NT
    mesh = plsc.VectorSubcoreMesh(core_axis_name="c", subcore_axis_name="s")

    @functools.partial(
        pl.kernel,
        out_type=jax.ShapeDtypeStruct((npad * H,), jnp.float32),
        mesh=mesh,
        compiler_params=pltpu.CompilerParams(needs_layout_passes=False),
        scratch_types=[
            pltpu.VMEM(((NT + 1) * H,), jnp.float32),
            pltpu.VMEM((SCH, H), jnp.float32),
            pltpu.VMEM((SCH, H), jnp.float32),
            pltpu.VMEM((SCH,), jnp.int32),
            pltpu.VMEM((SCH,), jnp.int32),
            pltpu.VMEM((48,), jnp.int32),
            pltpu.SemaphoreType.DMA,
            pltpu.SemaphoreType.DMA,
        ],
    )
    def scatter_k(m_hbm, ld_hbm, b_hbm, z_hbm, ztr_hbm, out_hbm,
                  acc_v, mrow0, mrow1, ld0, ld1, b_v, sm0, sm1):
        wid = lax.axis_index("s") * 2 + lax.axis_index("c")
        pltpu.sync_copy(z_hbm.at[pl.ds(wid * NT * H, NT * H)],
                        acc_v.at[pl.ds(0, NT * H)])
        pltpu.sync_copy(ztr_hbm, acc_v.at[pl.ds(NT * H, H)])
        lanes = lax.broadcasted_iota(jnp.int32, (16,), 0)
        cols = [lanes + 16 * j for j in range(H // 16)]
        mrows = [mrow0, mrow1]
        lds = [ld0, ld1]
        sems = [sm0, sm1]

        def _bound(pos):
            acc = jnp.zeros((), jnp.int32)
            for rg in range(3):
                v = b_v[pl.ds(16 * rg, 16)]
                msk = (lanes + 16 * rg == pos).astype(jnp.int32)
                acc = acc + jnp.sum(v * msk)
            return acc

        for k in range(k4):
            pltpu.sync_copy(b_hbm.at[pl.ds(k * 48, 48)], b_v)
            s = _bound(wid)
            e_ = _bound(wid + 1)
            b0 = (s // 8) * 8
            nch = (e_ - b0 + SCH - 1) // SCH

            def startc(i, buf, k=k, b0=b0):
                g0c = jnp.minimum(b0 + i * SCH, e - SCH)
                pltpu.async_copy(m_hbm.at[k, pl.ds(g0c, SCH)],
                                 mrows[buf], sems[buf])
                pltpu.async_copy(ld_hbm.at[pl.ds(k * e + g0c, SCH)],
                                 lds[buf], sems[buf])

            def compute(i, buf, s=s, e_=e_, b0=b0):
                pltpu.make_async_copy(m_hbm, mrows[buf], sems[buf]).wait()
                pltpu.make_async_copy(ld_hbm, lds[buf], sems[buf]).wait()
                g0 = b0 + i * SCH
                g0c = jnp.minimum(g0, e - SCH)
                lo = jnp.maximum(g0, s)
                mrow_v = mrows[buf]
                ld_v = lds[buf]

                def grp_body(b, carry2):
                    lvec = ld_v[pl.ds(16 * b, 16)]
                    gbase = g0c + 16 * b
                    for rr in range(16):
                        valid = (gbase + rr >= lo) & (gbase + rr < e_)
                        dsc = lax.gather(
                            lvec, jnp.full((16, 1), rr, jnp.int32),
                            lax.GatherDimensionNumbers(
                                offset_dims=(), collapsed_slice_dims=(0,),
                                start_index_map=(0,)),
                            slice_sizes=(1,),
                            mode=lax.GatherScatterMode.PROMISE_IN_BOUNDS)
                        dvec = jnp.where(valid, dsc, NT) * H
                        for j in range(H // 16):
                            plsc.addupdate_scatter(
                                acc_v, [dvec + cols[j]],
                                mrow_v[16 * b + rr, pl.ds(16 * j, 16)])
                    return carry2

                lax.fori_loop(0, SCH // 16, grp_body, 0)

            @pl.when(nch > 0)
            def _(startc=startc):
                startc(0, 0)

            def chunk_pair(ip, carry, nch=nch, startc=startc, compute=compute):
                for b in range(2):
                    i = 2 * ip + b

                    @pl.when(i + 1 < nch)
                    def _(i=i, b=b):
                        startc(i + 1, (b + 1) % 2)

                    @pl.when(i < nch)
                    def _(i=i, b=b):
                        compute(i, b)

                return carry

            lax.fori_loop(0, (nch + 1) // 2, chunk_pair, 0)

        pltpu.sync_copy(acc_v.at[pl.ds(0, NT * H)],
                        out_hbm.at[pl.ds(wid * NT * H, NT * H)])

    return scatter_k(m_all, ld_flat, bounds_flat, init_flat, ztr)


def _mm(a, b):
    return jnp.dot(a, b, preferred_element_type=jnp.float32)


def _node_enc_body(x_ref, w1_ref, b1_ref, w2_ref, b2_ref, o_ref):
    h1 = jnp.maximum(_mm(x_ref[...], w1_ref[...]) + b1_ref[...], 0.0)
    o_ref[...] = _mm(h1, w2_ref[...]) + b2_ref[...]


def _node_enc(x, p):
    n, f = x.shape
    r = 1024
    return pl.pallas_call(
        _node_enc_body,
        grid=(n // r,),
        in_specs=[
            pl.BlockSpec((r, f), lambda i: (i, 0)),
            pl.BlockSpec((f, H), lambda i: (0, 0)),
            pl.BlockSpec((1, H), lambda i: (0, 0)),
            pl.BlockSpec((H, H), lambda i: (0, 0)),
            pl.BlockSpec((1, H), lambda i: (0, 0)),
        ],
        out_specs=pl.BlockSpec((r, H), lambda i: (i, 0)),
        out_shape=jax.ShapeDtypeStruct((n, H), jnp.float32),
    )(x, p["l1"]["W"], p["l1"]["b"][None], p["l2"]["W"], p["l2"]["b"][None])


def _edge_enc_body(ea_ref, w1_ref, b1_ref, w2_ref, b2_ref, o_ref):
    h1 = jnp.maximum(_mm(ea_ref[0], w1_ref[0]) + b1_ref[0], 0.0)
    o_ref[0] = (_mm(h1, w2_ref[0]) + b2_ref[0]).astype(jnp.bfloat16)


def _edge_enc(ea_all, w1, b1, w2, b2):
    k, e, f = ea_all.shape
    be = 4000
    return pl.pallas_call(
        _edge_enc_body,
        grid=(k, e // be),
        in_specs=[
            pl.BlockSpec((1, be, f), lambda k_, i: (k_, i, 0)),
            pl.BlockSpec((1, f, H), lambda k_, i: (k_, 0, 0)),
            pl.BlockSpec((1, 1, H), lambda k_, i: (k_, 0, 0)),
            pl.BlockSpec((1, H, H), lambda k_, i: (k_, 0, 0)),
            pl.BlockSpec((1, 1, H), lambda k_, i: (k_, 0, 0)),
        ],
        out_specs=pl.BlockSpec((1, be, H), lambda k_, i: (k_, i, 0)),
        out_shape=jax.ShapeDtypeStruct((k, e, H), jnp.bfloat16),
    )(ea_all, w1, b1, w2, b2)


def _edge_mlp_body(hs_ref, hd_ref, e_ref, ws_ref, wd_ref, we_ref, b1_ref,
                   w2_ref, b2_ref, o_ref):
    hs = hs_ref[0].astype(jnp.bfloat16)
    hd = hd_ref[0].astype(jnp.bfloat16)
    pre = (_mm(hs, ws_ref[0]) + _mm(hd, wd_ref[0])
           + _mm(e_ref[0], we_ref[0]) + b1_ref[0])
    h1 = jnp.maximum(pre, 0.0).astype(jnp.bfloat16)
    o_ref[0] = _mm(h1, w2_ref[0]) + b2_ref[0]


def _edge_mlp(g_half, e_all, ws, wd, we, b1, w2, b2, k0):
    _, e, _ = e_all.shape
    kh = 2
    be = 4000
    h2 = 2 * H
    return pl.pallas_call(
        _edge_mlp_body,
        grid=(kh, e // be),
        in_specs=[
            pl.BlockSpec((1, be, H), lambda k_, i: (k_, i, 0)),
            pl.BlockSpec((1, be, H), lambda k_, i: (k_ + kh, i, 0)),
            pl.BlockSpec((1, be, H), lambda k_, i: (k0 + k_, i, 0)),
            pl.BlockSpec((1, H, h2), lambda k_, i: (k0 + k_, 0, 0)),
            pl.BlockSpec((1, H, h2), lambda k_, i: (k0 + k_, 0, 0)),
            pl.BlockSpec((1, H, h2), lambda k_, i: (k0 + k_, 0, 0)),
            pl.BlockSpec((1, 1, h2), lambda k_, i: (k0 + k_, 0, 0)),
            pl.BlockSpec((1, h2, H), lambda k_, i: (k0 + k_, 0, 0)),
            pl.BlockSpec((1, 1, H), lambda k_, i: (k0 + k_, 0, 0)),
        ],
        out_specs=pl.BlockSpec((1, be, H), lambda k_, i: (k_, i, 0)),
        out_shape=jax.ShapeDtypeStruct((kh, e, H), jnp.float32),
    )(g_half, g_half, e_all, ws, wd, we, b1, w2, b2)


def _node_upd_body(h_ref, agg_ref, wh_ref, wa_ref, b1_ref, w2_ref, b2_ref,
                   g_ref, bn_ref, o_ref):
    h = h_ref[...]
    pre = _mm(h, wh_ref[...]) + _mm(agg_ref[...], wa_ref[...]) + b1_ref[...]
    h1 = jnp.maximum(pre, 0.0)
    y = h + _mm(h1, w2_ref[...]) + b2_ref[...]
    mu = jnp.mean(y, axis=-1, keepdims=True)
    var = jnp.mean((y - mu) ** 2, axis=-1, keepdims=True)
    o_ref[...] = (y - mu) * jax.lax.rsqrt(var + 1e-5) * g_ref[...] + bn_ref[...]


def _node_update(h, agg, wh, wa, b1, w2, b2, g, bn):
    n = h.shape[0]
    r = 1024
    h2 = 2 * H
    return pl.pallas_call(
        _node_upd_body,
        grid=(n // r,),
        in_specs=[
            pl.BlockSpec((r, H), lambda i: (i, 0)),
            pl.BlockSpec((r, H), lambda i: (i, 0)),
            pl.BlockSpec((H, h2), lambda i: (0, 0)),
            pl.BlockSpec((H, h2), lambda i: (0, 0)),
            pl.BlockSpec((1, h2), lambda i: (0, 0)),
            pl.BlockSpec((h2, H), lambda i: (0, 0)),
            pl.BlockSpec((1, H), lambda i: (0, 0)),
            pl.BlockSpec((1, H), lambda i: (0, 0)),
            pl.BlockSpec((1, H), lambda i: (0, 0)),
        ],
        out_specs=pl.BlockSpec((r, H), lambda i: (i, 0)),
        out_shape=jax.ShapeDtypeStruct((n, H), jnp.float32),
    )(h, agg, wh, wa, b1, w2, b2, g, bn)


def _heads_body(n_real, h_ref, w1_ref, b1_ref, w2r_ref, b2_ref, np_ref, gm_ref):
    i = pl.program_id(0)
    h = h_ref[...]
    r = h.shape[0]
    h1 = jnp.maximum(_mm(h, w1_ref[...]) + b1_ref[...], 0.0)
    np_ref[...] = jnp.sum(h1 * w2r_ref[...], axis=-1, keepdims=True) + b2_ref[...]
    rid = lax.broadcasted_iota(jnp.int32, (r, 1), 0) + i * r
    bmax = jnp.max(jnp.where(rid < n_real, h, -jnp.inf), axis=0, keepdims=True)

    @pl.when(i == 0)
    def _():
        gm_ref[...] = bmax

    @pl.when(i > 0)
    def _():
        gm_ref[...] = jnp.maximum(gm_ref[...], bmax)


def _heads(h, p, n_real):
    n = h.shape[0]
    r = 1024
    hh = H // 2
    return pl.pallas_call(
        functools.partial(_heads_body, n_real),
        grid=(n // r,),
        in_specs=[
            pl.BlockSpec((r, H), lambda i: (i, 0)),
            pl.BlockSpec((H, hh), lambda i: (0, 0)),
            pl.BlockSpec((1, hh), lambda i: (0, 0)),
            pl.BlockSpec((1, hh), lambda i: (0, 0)),
            pl.BlockSpec((1, 1), lambda i: (0, 0)),
        ],
        out_specs=[
            pl.BlockSpec((r, 1), lambda i: (i, 0)),
            pl.BlockSpec((1, H), lambda i: (0, 0)),
        ],
        out_shape=[
            jax.ShapeDtypeStruct((n, 1), jnp.float32),
            jax.ShapeDtypeStruct((1, H), jnp.float32),
        ],
    )(h, p["l1"]["W"], p["l1"]["b"][None], p["l2"]["W"].T, p["l2"]["b"][None])


def _graph_head_body(g_ref, w1_ref, b1_ref, w2r_ref, b2_ref, o_ref):
    h1 = jnp.maximum(_mm(g_ref[...], w1_ref[...]) + b1_ref[...], 0.0)
    o_ref[...] = jnp.sum(h1 * w2r_ref[...], axis=-1, keepdims=True) + b2_ref[...]


def _graph_head(g, p):
    hh = H // 2
    return pl.pallas_call(
        _graph_head_body,
        grid=(1,),
        in_specs=[
            pl.BlockSpec((1, H), lambda i: (0, 0)),
            pl.BlockSpec((H, hh), lambda i: (0, 0)),
            pl.BlockSpec((1, hh), lambda i: (0, 0)),
            pl.BlockSpec((1, hh), lambda i: (0, 0)),
            pl.BlockSpec((1, 1), lambda i: (0, 0)),
        ],
        out_specs=pl.BlockSpec((1, 1), lambda i: (0, 0)),
        out_shape=jax.ShapeDtypeStruct((1, 1), jnp.float32),
    )(g, p["l1"]["W"], p["l1"]["b"][None], p["l2"]["W"].T, p["l2"]["b"][None])


def kernel(x, ei0, ei1, ei2, ei3, ea0, ea1, ea2, ea3, params):
    ei = [ei0, ei1, ei2, ei3]
    ea_all = jnp.stack([ea0, ea1, ea2, ea3])

    n = x.shape[0]
    npad = NW * NT
    x_pad = jnp.pad(x, ((0, npad - n), (0, 0)))
    h = _node_enc(x_pad, params["node_enc"])

    e = ei0.shape[1]
    src_all = jnp.stack([ee[0] for ee in ei])
    dst_all = jnp.stack([ee[1] for ee in ei])

    iota_e = jnp.arange(e, dtype=jnp.int32)
    (skey,) = lax.sort((dst_all * 131072 + iota_e[None],),
                       dimension=1, is_stable=False, num_keys=1)
    perm = skey & 131071
    dst_s = skey >> 17
    src_s = jnp.take_along_axis(src_all, perm, axis=1)
    ea_s = jnp.take_along_axis(
        jnp.reshape(ea_all, (K * e, -1)),
        jnp.reshape(perm + jnp.arange(K, dtype=jnp.int32)[:, None] * e,
                    (-1,))[:, None], axis=0)
    ea_s = jnp.reshape(ea_s, (K, e, -1))
    ld_flat = jnp.reshape(dst_s % NT, (-1,))
    targets = jnp.broadcast_to(jnp.arange(NW + 1, dtype=jnp.int32) * NT,
                               (K, NW + 1))
    bounds = jax.vmap(functools.partial(jnp.searchsorted, side="left"))(
        dst_s, targets).astype(jnp.int32)
    bounds_flat = jnp.reshape(
        jnp.pad(bounds, ((0, 0), (0, 48 - (NW + 1)))), (-1,))
    zeros_npad = jnp.zeros((npad * H,), jnp.float32)
    ztr = jnp.zeros((H,), jnp.float32)
    gidx_a = jnp.reshape(jnp.concatenate([src_s[:2], dst_s[:2]]), (-1,))
    gidx_b = jnp.reshape(jnp.concatenate([src_s[2:], dst_s[2:]]), (-1,))
    ld_a, ld_b = ld_flat[:2 * e], ld_flat[2 * e:]
    bounds_a, bounds_b = bounds_flat[:96], bounds_flat[96:]

    ew1 = jnp.stack([p["l1"]["W"] for p in params["edge_encs"]])
    eb1 = jnp.stack([p["l1"]["b"][None] for p in params["edge_encs"]])
    ew2 = jnp.stack([p["l2"]["W"] for p in params["edge_encs"]])
    eb2 = jnp.stack([p["l2"]["b"][None] for p in params["edge_encs"]])
    e_all = _edge_enc(ea_s, ew1, eb1, ew2, eb2)

    for lp in params["layers"]:
        ws = jnp.stack([p["l1"]["W"][:H]
                        for p in lp["edge_mlps"]]).astype(jnp.bfloat16)
        wd = jnp.stack([p["l1"]["W"][H:2 * H]
                        for p in lp["edge_mlps"]]).astype(jnp.bfloat16)
        we = jnp.stack([p["l1"]["W"][2 * H:]
                        for p in lp["edge_mlps"]]).astype(jnp.bfloat16)
        b1 = jnp.stack([p["l1"]["b"][None] for p in lp["edge_mlps"]])
        w2 = jnp.stack([p["l2"]["W"]
                        for p in lp["edge_mlps"]]).astype(jnp.bfloat16)
        b2 = jnp.stack([p["l2"]["b"][None] for p in lp["edge_mlps"]])

        g_a = _sc_gather(h, gidx_a, 4)
        m_a = _edge_mlp(g_a, e_all, ws, wd, we, b1, w2, b2, 0)
        g_b = _sc_gather(h, gidx_b, 4)
        m_b = _edge_mlp(g_b, e_all, ws, wd, we, b1, w2, b2, 2)
        agg_a = _sc_scatter(m_a, ld_a, bounds_a, zeros_npad, ztr)
        agg_b = _sc_scatter(m_b, ld_b, bounds_b, agg_a, ztr)
        agg_pad = jnp.reshape(agg_b, (npad, H))

        nm = lp["node_mlp"]
        h = _node_update(
            h, agg_pad,
            nm["l1"]["W"][:H], nm["l1"]["W"][H:], nm["l1"]["b"][None],
            nm["l2"]["W"], nm["l2"]["b"][None],
            lp["norm"]["g"][None], lp["norm"]["b"][None])

    node_pred, gmax = _heads(h, params["reg_head"], n)
    graph_pred = _graph_head(gmax, params["graph_head"])
    return (jnp.reshape(node_pred, (-1,))[:n], jnp.reshape(graph_pred, (1,)))
